# Initial kernel scaffold; baseline (speedup 1.0000x reference)
import jax
import jax.numpy as jnp
from jax.experimental import pallas as pl


def kernel(x, edge_index, W1, b1, W2, b2, W3, b3, Wout, bout):
    raise NotImplementedError("write your pallas kernel here")


# speedup vs baseline: 11.2690x; 11.2690x over previous
import functools
import math

import jax
import jax.numpy as jnp
from jax import lax
from jax.experimental import pallas as pl
from jax.experimental.pallas import tpu as pltpu
from jax.experimental.pallas import tpu_sc as plsc

NC = 2
NS = 16
LANES = 16
CHUNK = 128

_HI = lax.Precision.HIGHEST
_F32 = jnp.float32


def _row_split(n):
    for mult in (80, 40, 16, 8):
        span = -(-(-(-n // NS)) // mult) * mult
        tail = n - (NS - 1) * span
        if 0 < tail <= span and tail % 8 == 0:
            return span, tail, math.gcd(span, tail)
    raise ValueError(f"no 8-aligned row split for n={n}")


def _fill2d(ref, rows, width, val):
    @pl.loop(0, rows)
    def _(i):
        @pl.loop(0, width, step=LANES)
        def _(j):
            ref[i, pl.ds(j, LANES)] = jnp.full((LANES,), val, _F32)


def _sc_deg(col, n):
    e = col.shape[0]
    per = e // (NC * NS)
    assert per * NC * NS == e and per % 8 == 0
    nf, rem = divmod(per, CHUNK)
    span, tail, zch = _row_split(n)

    @functools.partial(
        pl.kernel,
        out_type=jax.ShapeDtypeStruct((2 * n, 16), _F32),
        mesh=plsc.VectorSubcoreMesh(core_axis_name="c", subcore_axis_name="s"),
        scratch_types=[
            pltpu.VMEM((CHUNK, 16), _F32),
            pltpu.VMEM((max(rem, 8), 16), _F32),
            pltpu.VMEM((CHUNK,), jnp.int32),
            pltpu.VMEM((max(rem, 8),), jnp.int32),
            pltpu.VMEM((zch, 16), _F32),
            pltpu.VMEM_SHARED((n, 16), _F32),
        ],
        compiler_params=pltpu.CompilerParams(use_tc_tiling_on_sc=False),
    )
    def deg_kernel(col_hbm, out_hbm, ones_v, ones_r, cidx, cidx_r, zbuf, acc):
        cid = lax.axis_index("c")
        sid = lax.axis_index("s")
        _fill2d(ones_v, CHUNK, 16, 1.0)
        if rem:
            _fill2d(ones_r, rem, 16, 1.0)
        _fill2d(zbuf, zch, 16, 0.0)
        zb = sid * span
        nzc = jnp.minimum(span, n - zb) // zch

        @pl.loop(0, nzc)
        def _(r):
            pltpu.sync_copy(zbuf, acc.at[pl.ds(zb + r * zch, zch)])

        plsc.subcore_barrier()
        eb = (cid * NS + sid) * per

        @pl.loop(0, nf)
        def _(k):
            pltpu.sync_copy(col_hbm.at[pl.ds(eb + k * CHUNK, CHUNK)], cidx)
            pltpu.sync_copy(ones_v, acc.at[cidx], add=True)

        if rem:
            pltpu.sync_copy(col_hbm.at[pl.ds(eb + nf * CHUNK, rem)], cidx_r)
            pltpu.sync_copy(ones_r, acc.at[cidx_r], add=True)

        plsc.subcore_barrier()

        @pl.loop(0, nzc)
        def _(r):
            pltpu.sync_copy(acc.at[pl.ds(zb + r * zch, zch)],
                            out_hbm.at[pl.ds(cid * n + zb + r * zch, zch)])

    return deg_kernel(col)


def _sc_agg(rows2, col, g, n, hf):
    e = col.shape[0]
    per = e // NS
    assert per * NS == e and per % 8 == 0
    nf, rem = divmod(per, CHUNK)
    assert hf % LANES == 0
    span, tail, zch = _row_split(n)

    @functools.partial(
        pl.kernel,
        out_type=jax.ShapeDtypeStruct((2 * n, hf), _F32),
        mesh=plsc.VectorSubcoreMesh(core_axis_name="c", subcore_axis_name="s"),
        scratch_types=[
            pltpu.VMEM((CHUNK,), jnp.int32),
            pltpu.VMEM((CHUNK,), jnp.int32),
            pltpu.VMEM((CHUNK, hf), _F32),
            pltpu.VMEM((max(rem, 8),), jnp.int32),
            pltpu.VMEM((max(rem, 8),), jnp.int32),
            pltpu.VMEM((max(rem, 8), hf), _F32),
            pltpu.VMEM((zch, hf), _F32),
            pltpu.VMEM_SHARED((n, hf), _F32),
        ],
        compiler_params=pltpu.CompilerParams(use_tc_tiling_on_sc=False),
    )
    def agg_kernel(rows_hbm, col_hbm, g_hbm, s_hbm,
                   ridx, cidx, gbuf, ridx_r, cidx_r, gbuf_r, zbuf, acc):
        cid = lax.axis_index("c")
        sid = lax.axis_index("s")
        _fill2d(zbuf, zch, hf, 0.0)
        zb = sid * span
        nzc = jnp.minimum(span, n - zb) // zch

        @pl.loop(0, nzc)
        def _(r):
            pltpu.sync_copy(zbuf, acc.at[pl.ds(zb + r * zch, zch)])

        plsc.subcore_barrier()
        eb = sid * per

        @pl.loop(0, nf)
        def _(k):
            b = eb + k * CHUNK
            pltpu.sync_copy(rows_hbm.at[pl.ds(cid * e + b, CHUNK)], ridx)
            pltpu.sync_copy(col_hbm.at[pl.ds(b, CHUNK)], cidx)
            pltpu.sync_copy(g_hbm.at[ridx], gbuf)
            pltpu.sync_copy(gbuf, acc.at[cidx], add=True)

        if rem:
            b = eb + nf * CHUNK
            pltpu.sync_copy(rows_hbm.at[pl.ds(cid * e + b, rem)], ridx_r)
            pltpu.sync_copy(col_hbm.at[pl.ds(b, rem)], cidx_r)
            pltpu.sync_copy(g_hbm.at[ridx_r], gbuf_r)
            pltpu.sync_copy(gbuf_r, acc.at[cidx_r], add=True)

        plsc.subcore_barrier()

        @pl.loop(0, nzc)
        def _(r):
            pltpu.sync_copy(acc.at[pl.ds(zb + r * zch, zch)],
                            s_hbm.at[pl.ds(cid * n + zb + r * zch, zch)])

    return agg_kernel(rows2, col, g)


_BN = 400


def _tc_prolog(x, degp, n):
    din = x.shape[1]
    h = din // 2
    nb = n // _BN

    def body(x_ref, d0, d1, g_ref, dinv_ref):
        dinv = lax.rsqrt(d0[:, 0:1] + d1[:, 0:1] + 1.0)
        g = x_ref[...] * dinv
        g_ref[0] = g[:, :h]
        g_ref[1] = g[:, h:]
        dinv_ref[...] = dinv

    return pl.pallas_call(
        body,
        grid=(nb,),
        in_specs=[
            pl.BlockSpec((_BN, din), lambda i: (i, 0)),
            pl.BlockSpec((_BN, 16), lambda i: (i, 0)),
            pl.BlockSpec((_BN, 16), lambda i: (i + nb, 0)),
        ],
        out_specs=[
            pl.BlockSpec((2, _BN, h), lambda i: (0, i, 0)),
            pl.BlockSpec((_BN, 1), lambda i: (i, 0)),
        ],
        out_shape=[
            jax.ShapeDtypeStruct((2, n, h), _F32),
            jax.ShapeDtypeStruct((n, 1), _F32),
        ],
    )(x, degp, degp)


def _tc_layer(s, g, dinv, w, b, n, w2=None):
    hf = s.shape[1]
    dout = (w2 if w2 is not None else w).shape[1]
    ho = dout // 2
    nb = n // _BN

    def body(s0, s1, g0, g1, dv, w_ref, b_ref, *rest):
        if w2 is not None:
            w2_ref, out = rest
        else:
            (out,) = rest
        t = jnp.concatenate([s0[...] + g0[...], s1[...] + g1[...]], axis=1)
        t = t * dv[...]
        hh = jnp.dot(t, w_ref[...], precision=_HI,
                     preferred_element_type=_F32) + b_ref[...]
        hh = jnp.maximum(hh, 0.0)
        if w2 is not None:
            hh = jnp.dot(hh, w2_ref[...], precision=_HI,
                         preferred_element_type=_F32)
        gg = hh * dv[...]
        out[0] = gg[:, :ho]
        out[1] = gg[:, ho:]

    in_specs = [
        pl.BlockSpec((_BN, hf), lambda i: (i, 0)),
        pl.BlockSpec((_BN, hf), lambda i: (i + nb, 0)),
        pl.BlockSpec((_BN, hf), lambda i: (i, 0)),
        pl.BlockSpec((_BN, hf), lambda i: (i + nb, 0)),
        pl.BlockSpec((_BN, 1), lambda i: (i, 0)),
        pl.BlockSpec(w.shape, lambda i: (0, 0)),
        pl.BlockSpec(b.shape, lambda i: (0, 0)),
    ]
    args = [s, s, g, g, dinv, w, b]
    if w2 is not None:
        in_specs.append(pl.BlockSpec(w2.shape, lambda i: (0, 0)))
        args.append(w2)
    return pl.pallas_call(
        body,
        grid=(nb,),
        in_specs=in_specs,
        out_specs=pl.BlockSpec((2, _BN, ho), lambda i: (0, i, 0)),
        out_shape=jax.ShapeDtypeStruct((2, n, ho), _F32),
    )(*args)


def _tc_epilog(s, g, dinv, b34, n):
    hf = s.shape[1]
    nb = n // _BN

    def body(s0, s1, g0, g1, dv, b_ref, out):
        t = jnp.concatenate([s0[...] + g0[...], s1[...] + g1[...]], axis=1)
        out[...] = t * dv[...] + b_ref[...]

    return pl.pallas_call(
        body,
        grid=(nb,),
        in_specs=[
            pl.BlockSpec((_BN, hf), lambda i: (i, 0)),
            pl.BlockSpec((_BN, hf), lambda i: (i + nb, 0)),
            pl.BlockSpec((_BN, hf), lambda i: (i, 0)),
            pl.BlockSpec((_BN, hf), lambda i: (i + nb, 0)),
            pl.BlockSpec((_BN, 1), lambda i: (i, 0)),
            pl.BlockSpec(b34.shape, lambda i: (0, 0)),
        ],
        out_specs=pl.BlockSpec((_BN, 2 * hf), lambda i: (i, 0)),
        out_shape=jax.ShapeDtypeStruct((n, 2 * hf), _F32),
    )(s, s, g, g, dinv, b34)


def _tc_fold(w3, wout, b3, bout):
    def body(w3_ref, wo_ref, b3_ref, bo_ref, w34_ref, b34_ref):
        w34_ref[...] = jnp.dot(w3_ref[...], wo_ref[...], precision=_HI,
                               preferred_element_type=_F32)
        b34_ref[...] = jnp.dot(b3_ref[...], wo_ref[...], precision=_HI,
                               preferred_element_type=_F32) + bo_ref[...]

    return pl.pallas_call(
        body,
        out_shape=[
            jax.ShapeDtypeStruct((w3.shape[0], wout.shape[1]), _F32),
            jax.ShapeDtypeStruct((1, wout.shape[1]), _F32),
        ],
    )(w3, wout, b3, bout)


def kernel(x, edge_index, W1, b1, W2, b2, W3, b3, Wout, bout):
    n, din = x.shape
    e = edge_index.shape[1]
    row = edge_index[0]
    col = edge_index[1]
    rows2 = jnp.concatenate([row, row + jnp.int32(n)])

    b1r = b1.reshape(1, -1)
    b2r = b2.reshape(1, -1)
    b3r = b3.reshape(1, -1)
    boutr = bout.reshape(1, -1)

    w34, b34 = _tc_fold(W3, Wout, b3r, boutr)

    degp = _sc_deg(col, n)
    g1_3d, dinv = _tc_prolog(x, degp, n)
    g1 = g1_3d.reshape(2 * n, din // 2)
    s1 = _sc_agg(rows2, col, g1, n, din // 2)

    g2_3d = _tc_layer(s1, g1, dinv, W1, b1r, n)
    g2 = g2_3d.reshape(2 * n, W1.shape[1] // 2)
    s2 = _sc_agg(rows2, col, g2, n, W1.shape[1] // 2)

    g3_3d = _tc_layer(s2, g2, dinv, W2, b2r, n, w2=w34)
    g3 = g3_3d.reshape(2 * n, Wout.shape[1] // 2)
    s3 = _sc_agg(rows2, col, g3, n, Wout.shape[1] // 2)

    return _tc_epilog(s3, g3, dinv, b34, n)


# speedup vs baseline: 19.8287x; 1.7596x over previous
import functools
import math

import jax
import jax.numpy as jnp
from jax import lax
from jax.experimental import pallas as pl
from jax.experimental.pallas import tpu as pltpu
from jax.experimental.pallas import tpu_sc as plsc

NC = 2
NS = 16
LANES = 16
CHUNK = 128

_HI = lax.Precision.HIGHEST
_F32 = jnp.float32


def _row_split(n):
    for mult in (80, 40, 16, 8):
        span = -(-(-(-n // NS)) // mult) * mult
        tail = n - (NS - 1) * span
        if 0 < tail <= span and tail % 8 == 0:
            return span, tail, math.gcd(span, tail)
    raise ValueError(f"no 8-aligned row split for n={n}")


def _fill2d(ref, rows, width, val):
    @pl.loop(0, rows)
    def _(i):
        @pl.loop(0, width, step=LANES)
        def _(j):
            ref[i, pl.ds(j, LANES)] = jnp.full((LANES,), val, _F32)


NB = 4


def _chunk_split(nchunks, nworkers):
    base, extra = divmod(nchunks, nworkers)
    return base, extra


def _worker_chunks(wid, base, extra):
    start = wid * base + jnp.minimum(wid, extra)
    cnt = base + jnp.where(wid < extra, 1, 0)
    return start, cnt


def _sc_deg(col, n):
    nchunks = col.shape[0] // CHUNK
    span, tail, zch = _row_split(n)
    base, extra = _chunk_split(nchunks, NC * NS)

    @functools.partial(
        pl.kernel,
        out_type=jax.ShapeDtypeStruct((2 * n, 16), _F32),
        mesh=plsc.VectorSubcoreMesh(core_axis_name="c", subcore_axis_name="s"),
        scratch_types=[
            pltpu.VMEM((CHUNK, 16), _F32),
            pltpu.VMEM((NB, CHUNK), jnp.int32),
            pltpu.VMEM((zch, 16), _F32),
            pltpu.VMEM_SHARED((n, 16), _F32),
            pltpu.SemaphoreType.DMA((NB,)),
            pltpu.SemaphoreType.DMA((NB,)),
        ],
        compiler_params=pltpu.CompilerParams(use_tc_tiling_on_sc=False),
    )
    def deg_kernel(col_hbm, out_hbm, ones_v, cidx, zbuf, acc, sem_i, sem_s):
        cid = lax.axis_index("c")
        sid = lax.axis_index("s")
        _fill2d(ones_v, CHUNK, 16, 1.0)
        _fill2d(zbuf, zch, 16, 0.0)
        zb = sid * span
        nzc = jnp.minimum(span, n - zb) // zch

        @pl.loop(0, nzc)
        def _(r):
            pltpu.sync_copy(zbuf, acc.at[pl.ds(zb + r * zch, zch)])

        plsc.subcore_barrier()
        cstart, ccnt = _worker_chunks(cid * NS + sid, base, extra)
        ngrp = ccnt // NB

        @pl.loop(0, ngrp)
        def _(g):
            cb = (cstart + g * NB) * CHUNK
            ld = [pltpu.async_copy(col_hbm.at[pl.ds(cb + b * CHUNK, CHUNK)],
                                   cidx.at[b], sem_i.at[b]) for b in range(NB)]
            st = []
            for b in range(NB):
                ld[b].wait()
                st.append(pltpu.async_copy(ones_v, acc.at[cidx.at[b]],
                                           sem_s.at[b], add=True))
            for b in range(NB):
                st[b].wait()

        @pl.loop(ngrp * NB, ccnt)
        def _(k):
            pltpu.sync_copy(col_hbm.at[pl.ds((cstart + k) * CHUNK, CHUNK)],
                            cidx.at[0])
            pltpu.sync_copy(ones_v, acc.at[cidx.at[0]], add=True)

        plsc.subcore_barrier()

        @pl.loop(0, nzc)
        def _(r):
            pltpu.sync_copy(acc.at[pl.ds(zb + r * zch, zch)],
                            out_hbm.at[pl.ds(cid * n + zb + r * zch, zch)])

    return deg_kernel(col)


def _sc_agg(rows2, col, g, n, hf, nb=NB):
    e = col.shape[0]
    nchunks = e // CHUNK
    assert hf % LANES == 0
    span, tail, zch = _row_split(n)
    while zch * hf > 5120 and zch % 16 == 0:
        zch //= 2
    base, extra = _chunk_split(nchunks, NS)

    @functools.partial(
        pl.kernel,
        out_type=jax.ShapeDtypeStruct((2 * n, hf), _F32),
        mesh=plsc.VectorSubcoreMesh(core_axis_name="c", subcore_axis_name="s"),
        scratch_types=[
            pltpu.VMEM((nb, CHUNK), jnp.int32),
            pltpu.VMEM((nb, CHUNK), jnp.int32),
            pltpu.VMEM((nb, CHUNK, hf), _F32),
            pltpu.VMEM((zch, hf), _F32),
            pltpu.VMEM_SHARED((n, hf), _F32),
            pltpu.SemaphoreType.DMA((nb,)),
            pltpu.SemaphoreType.DMA((nb,)),
            pltpu.SemaphoreType.DMA((nb,)),
            pltpu.SemaphoreType.DMA((nb,)),
        ],
        compiler_params=pltpu.CompilerParams(use_tc_tiling_on_sc=False),
    )
    def agg_kernel(rows_hbm, col_hbm, g_hbm, s_hbm,
                   ridx, cidx, gbuf, zbuf, acc, sem_ir, sem_ic, sem_g, sem_s):
        cid = lax.axis_index("c")
        sid = lax.axis_index("s")
        _fill2d(zbuf, zch, hf, 0.0)
        zb = sid * span
        nzc = jnp.minimum(span, n - zb) // zch

        @pl.loop(0, nzc)
        def _(r):
            pltpu.sync_copy(zbuf, acc.at[pl.ds(zb + r * zch, zch)])

        plsc.subcore_barrier()
        cstart, ccnt = _worker_chunks(sid, base, extra)
        ngrp = ccnt // nb

        @pl.loop(0, ngrp)
        def _(grp):
            cb = (cstart + grp * nb) * CHUNK
            ldr = [pltpu.async_copy(
                rows_hbm.at[pl.ds(cid * e + cb + b * CHUNK, CHUNK)],
                ridx.at[b], sem_ir.at[b]) for b in range(nb)]
            ldc = [pltpu.async_copy(
                col_hbm.at[pl.ds(cb + b * CHUNK, CHUNK)],
                cidx.at[b], sem_ic.at[b]) for b in range(nb)]
            gth = []
            for b in range(nb):
                ldr[b].wait()
                gth.append(pltpu.async_copy(g_hbm.at[ridx.at[b]], gbuf.at[b],
                                            sem_g.at[b]))
            sct = []
            for b in range(nb):
                gth[b].wait()
                ldc[b].wait()
                sct.append(pltpu.async_copy(gbuf.at[b], acc.at[cidx.at[b]],
                                            sem_s.at[b], add=True))
            for b in range(nb):
                sct[b].wait()

        @pl.loop(ngrp * nb, ccnt)
        def _(k):
            kb = (cstart + k) * CHUNK
            pltpu.sync_copy(rows_hbm.at[pl.ds(cid * e + kb, CHUNK)],
                            ridx.at[0])
            pltpu.sync_copy(col_hbm.at[pl.ds(kb, CHUNK)], cidx.at[0])
            pltpu.sync_copy(g_hbm.at[ridx.at[0]], gbuf.at[0])
            pltpu.sync_copy(gbuf.at[0], acc.at[cidx.at[0]], add=True)

        plsc.subcore_barrier()

        @pl.loop(0, nzc)
        def _(r):
            pltpu.sync_copy(acc.at[pl.ds(zb + r * zch, zch)],
                            s_hbm.at[pl.ds(cid * n + zb + r * zch, zch)])

    return agg_kernel(rows2, col, g)


_BN = 400


def _tc_prolog(x, degp, n):
    din = x.shape[1]
    h = din // 2
    nb = n // _BN

    def body(x_ref, d0, d1, g_ref, dinv_ref):
        dinv = lax.rsqrt(d0[:, 0:1] + d1[:, 0:1] + 1.0)
        g = x_ref[...] * dinv
        g_ref[0] = g[:, :h]
        g_ref[1] = g[:, h:]
        dinv_ref[...] = dinv

    return pl.pallas_call(
        body,
        grid=(nb,),
        in_specs=[
            pl.BlockSpec((_BN, din), lambda i: (i, 0)),
            pl.BlockSpec((_BN, 16), lambda i: (i, 0)),
            pl.BlockSpec((_BN, 16), lambda i: (i + nb, 0)),
        ],
        out_specs=[
            pl.BlockSpec((2, _BN, h), lambda i: (0, i, 0)),
            pl.BlockSpec((_BN, 1), lambda i: (i, 0)),
        ],
        out_shape=[
            jax.ShapeDtypeStruct((2, n, h), _F32),
            jax.ShapeDtypeStruct((n, 1), _F32),
        ],
    )(x, degp, degp)


def _tc_layer(s, g, dinv, w, b, n, w2=None):
    hf = s.shape[1]
    dout = (w2 if w2 is not None else w).shape[1]
    ho = dout // 2
    nb = n // _BN

    def body(s0, s1, g0, g1, dv, w_ref, b_ref, *rest):
        if w2 is not None:
            w2_ref, out = rest
        else:
            (out,) = rest
        t = jnp.concatenate([s0[...] + g0[...], s1[...] + g1[...]], axis=1)
        t = t * dv[...]
        hh = jnp.dot(t, w_ref[...], precision=_HI,
                     preferred_element_type=_F32) + b_ref[...]
        hh = jnp.maximum(hh, 0.0)
        if w2 is not None:
            hh = jnp.dot(hh, w2_ref[...], precision=_HI,
                         preferred_element_type=_F32)
        gg = hh * dv[...]
        out[0] = gg[:, :ho]
        out[1] = gg[:, ho:]

    in_specs = [
        pl.BlockSpec((_BN, hf), lambda i: (i, 0)),
        pl.BlockSpec((_BN, hf), lambda i: (i + nb, 0)),
        pl.BlockSpec((_BN, hf), lambda i: (i, 0)),
        pl.BlockSpec((_BN, hf), lambda i: (i + nb, 0)),
        pl.BlockSpec((_BN, 1), lambda i: (i, 0)),
        pl.BlockSpec(w.shape, lambda i: (0, 0)),
        pl.BlockSpec(b.shape, lambda i: (0, 0)),
    ]
    args = [s, s, g, g, dinv, w, b]
    if w2 is not None:
        in_specs.append(pl.BlockSpec(w2.shape, lambda i: (0, 0)))
        args.append(w2)
    return pl.pallas_call(
        body,
        grid=(nb,),
        in_specs=in_specs,
        out_specs=pl.BlockSpec((2, _BN, ho), lambda i: (0, i, 0)),
        out_shape=jax.ShapeDtypeStruct((2, n, ho), _F32),
    )(*args)


def _tc_epilog(s, g, dinv, b34, n):
    hf = s.shape[1]
    nb = n // _BN

    def body(s0, s1, g0, g1, dv, b_ref, out):
        t = jnp.concatenate([s0[...] + g0[...], s1[...] + g1[...]], axis=1)
        out[...] = t * dv[...] + b_ref[...]

    return pl.pallas_call(
        body,
        grid=(nb,),
        in_specs=[
            pl.BlockSpec((_BN, hf), lambda i: (i, 0)),
            pl.BlockSpec((_BN, hf), lambda i: (i + nb, 0)),
            pl.BlockSpec((_BN, hf), lambda i: (i, 0)),
            pl.BlockSpec((_BN, hf), lambda i: (i + nb, 0)),
            pl.BlockSpec((_BN, 1), lambda i: (i, 0)),
            pl.BlockSpec(b34.shape, lambda i: (0, 0)),
        ],
        out_specs=pl.BlockSpec((_BN, 2 * hf), lambda i: (i, 0)),
        out_shape=jax.ShapeDtypeStruct((n, 2 * hf), _F32),
    )(s, s, g, g, dinv, b34)


def _tc_fold(w3, wout, b3, bout):
    def body(w3_ref, wo_ref, b3_ref, bo_ref, w34_ref, b34_ref):
        w34_ref[...] = jnp.dot(w3_ref[...], wo_ref[...], precision=_HI,
                               preferred_element_type=_F32)
        b34_ref[...] = jnp.dot(b3_ref[...], wo_ref[...], precision=_HI,
                               preferred_element_type=_F32) + bo_ref[...]

    return pl.pallas_call(
        body,
        out_shape=[
            jax.ShapeDtypeStruct((w3.shape[0], wout.shape[1]), _F32),
            jax.ShapeDtypeStruct((1, wout.shape[1]), _F32),
        ],
    )(w3, wout, b3, bout)


def kernel(x, edge_index, W1, b1, W2, b2, W3, b3, Wout, bout):
    n, din = x.shape
    e = edge_index.shape[1]
    assert e % CHUNK == 0
    row = edge_index[0]
    col = edge_index[1]
    rows2 = jnp.concatenate([row, row + jnp.int32(n)])

    b1r = b1.reshape(1, -1)
    b2r = b2.reshape(1, -1)
    b3r = b3.reshape(1, -1)
    boutr = bout.reshape(1, -1)

    w34, b34 = _tc_fold(W3, Wout, b3r, boutr)

    degp = _sc_deg(col, n)
    g1_3d, dinv = _tc_prolog(x, degp, n)
    g1 = g1_3d.reshape(2 * n, din // 2)
    s1 = _sc_agg(rows2, col, g1, n, din // 2)

    g2_3d = _tc_layer(s1, g1, dinv, W1, b1r, n)
    g2 = g2_3d.reshape(2 * n, W1.shape[1] // 2)
    s2 = _sc_agg(rows2, col, g2, n, W1.shape[1] // 2, nb=2)

    g3_3d = _tc_layer(s2, g2, dinv, W2, b2r, n, w2=w34)
    g3 = g3_3d.reshape(2 * n, Wout.shape[1] // 2)
    s3 = _sc_agg(rows2, col, g3, n, Wout.shape[1] // 2)

    return _tc_epilog(s3, g3, dinv, b34, n)


# speedup vs baseline: 20.5585x; 1.0368x over previous
import functools
import math

import jax
import jax.numpy as jnp
from jax import lax
from jax.experimental import pallas as pl
from jax.experimental.pallas import tpu as pltpu
from jax.experimental.pallas import tpu_sc as plsc

NC = 2
NS = 16
LANES = 16
CHUNK = 128

_HI = lax.Precision.HIGHEST
_F32 = jnp.float32


def _row_split(n):
    for mult in (80, 40, 16, 8):
        span = -(-(-(-n // NS)) // mult) * mult
        tail = n - (NS - 1) * span
        if 0 < tail <= span and tail % 8 == 0:
            return span, tail, math.gcd(span, tail)
    raise ValueError(f"no 8-aligned row split for n={n}")


def _fill2d(ref, rows, width, val):
    @pl.loop(0, rows)
    def _(i):
        @pl.loop(0, width, step=LANES)
        def _(j):
            ref[i, pl.ds(j, LANES)] = jnp.full((LANES,), val, _F32)


NB = 4


def _chunk_split(nchunks, nworkers):
    base, extra = divmod(nchunks, nworkers)
    return base, extra


def _worker_chunks(wid, base, extra):
    start = wid * base + jnp.minimum(wid, extra)
    cnt = base + jnp.where(wid < extra, 1, 0)
    return start, cnt


def _sc_deg(col, n):
    ck = 2 * CHUNK
    nchunks = col.shape[0] // ck
    assert nchunks * ck == col.shape[0]
    span, tail, zch = _row_split(n)
    base, extra = _chunk_split(nchunks, NC * NS)

    @functools.partial(
        pl.kernel,
        out_type=jax.ShapeDtypeStruct((2 * n, 16), _F32),
        mesh=plsc.VectorSubcoreMesh(core_axis_name="c", subcore_axis_name="s"),
        scratch_types=[
            pltpu.VMEM((ck, 16), _F32),
            pltpu.VMEM((NB, ck), jnp.int32),
            pltpu.VMEM((zch, 16), _F32),
            pltpu.VMEM_SHARED((n, 16), _F32),
            pltpu.SemaphoreType.DMA((NB,)),
            pltpu.SemaphoreType.DMA((NB,)),
        ],
        compiler_params=pltpu.CompilerParams(use_tc_tiling_on_sc=False),
    )
    def deg_kernel(col_hbm, out_hbm, ones_v, cidx, zbuf, acc, sem_i, sem_s):
        cid = lax.axis_index("c")
        sid = lax.axis_index("s")
        _fill2d(ones_v, ck, 16, 1.0)
        _fill2d(zbuf, zch, 16, 0.0)
        zb = sid * span
        nzc = jnp.minimum(span, n - zb) // zch

        @pl.loop(0, nzc)
        def _(r):
            pltpu.sync_copy(zbuf, acc.at[pl.ds(zb + r * zch, zch)])

        plsc.subcore_barrier()
        cstart, ccnt = _worker_chunks(cid * NS + sid, base, extra)
        ngrp = ccnt // NB

        @pl.loop(0, ngrp)
        def _(g):
            cb = (cstart + g * NB) * ck
            ld = [pltpu.async_copy(col_hbm.at[pl.ds(cb + b * ck, ck)],
                                   cidx.at[b], sem_i.at[b]) for b in range(NB)]
            st = []
            for b in range(NB):
                ld[b].wait()
                st.append(pltpu.async_copy(ones_v, acc.at[cidx.at[b]],
                                           sem_s.at[b], add=True))
            for b in range(NB):
                st[b].wait()

        @pl.loop(ngrp * NB, ccnt)
        def _(k):
            pltpu.sync_copy(col_hbm.at[pl.ds((cstart + k) * ck, ck)],
                            cidx.at[0])
            pltpu.sync_copy(ones_v, acc.at[cidx.at[0]], add=True)

        plsc.subcore_barrier()

        @pl.loop(0, nzc)
        def _(r):
            pltpu.sync_copy(acc.at[pl.ds(zb + r * zch, zch)],
                            out_hbm.at[pl.ds(cid * n + zb + r * zch, zch)])

    return deg_kernel(col)


def _sc_agg(rows2, col, g, n, hf, nb=NB, ck=2 * CHUNK):
    e = col.shape[0]
    nchunks = e // ck
    assert nchunks * ck == e and hf % LANES == 0
    span, tail, zch = _row_split(n)
    while zch * hf > 5120 and zch % 16 == 0:
        zch //= 2
    base, extra = _chunk_split(nchunks, NS)

    @functools.partial(
        pl.kernel,
        out_type=jax.ShapeDtypeStruct((2 * n, hf), _F32),
        mesh=plsc.VectorSubcoreMesh(core_axis_name="c", subcore_axis_name="s"),
        scratch_types=[
            pltpu.VMEM((nb, ck), jnp.int32),
            pltpu.VMEM((nb, ck), jnp.int32),
            pltpu.VMEM((nb, ck, hf), _F32),
            pltpu.VMEM((zch, hf), _F32),
            pltpu.VMEM_SHARED((n, hf), _F32),
            pltpu.SemaphoreType.DMA((nb,)),
            pltpu.SemaphoreType.DMA((nb,)),
            pltpu.SemaphoreType.DMA((nb,)),
            pltpu.SemaphoreType.DMA((nb,)),
        ],
        compiler_params=pltpu.CompilerParams(use_tc_tiling_on_sc=False),
    )
    def agg_kernel(rows_hbm, col_hbm, g_hbm, s_hbm,
                   ridx, cidx, gbuf, zbuf, acc, sem_ir, sem_ic, sem_g, sem_s):
        cid = lax.axis_index("c")
        sid = lax.axis_index("s")
        _fill2d(zbuf, zch, hf, 0.0)
        zb = sid * span
        nzc = jnp.minimum(span, n - zb) // zch

        @pl.loop(0, nzc)
        def _(r):
            pltpu.sync_copy(zbuf, acc.at[pl.ds(zb + r * zch, zch)])

        plsc.subcore_barrier()
        cstart, ccnt = _worker_chunks(sid, base, extra)
        ngrp = ccnt // nb

        @pl.loop(0, ngrp)
        def _(grp):
            cb = (cstart + grp * nb) * ck
            ldr = [pltpu.async_copy(
                rows_hbm.at[pl.ds(cid * e + cb + b * ck, ck)],
                ridx.at[b], sem_ir.at[b]) for b in range(nb)]
            ldc = [pltpu.async_copy(
                col_hbm.at[pl.ds(cb + b * ck, ck)],
                cidx.at[b], sem_ic.at[b]) for b in range(nb)]
            gth = []
            for b in range(nb):
                ldr[b].wait()
                gth.append(pltpu.async_copy(g_hbm.at[ridx.at[b]], gbuf.at[b],
                                            sem_g.at[b]))
            sct = []
            for b in range(nb):
                gth[b].wait()
                ldc[b].wait()
                sct.append(pltpu.async_copy(gbuf.at[b], acc.at[cidx.at[b]],
                                            sem_s.at[b], add=True))
            for b in range(nb):
                sct[b].wait()

        @pl.loop(ngrp * nb, ccnt)
        def _(k):
            kb = (cstart + k) * ck
            pltpu.sync_copy(rows_hbm.at[pl.ds(cid * e + kb, ck)],
                            ridx.at[0])
            pltpu.sync_copy(col_hbm.at[pl.ds(kb, ck)], cidx.at[0])
            pltpu.sync_copy(g_hbm.at[ridx.at[0]], gbuf.at[0])
            pltpu.sync_copy(gbuf.at[0], acc.at[cidx.at[0]], add=True)

        plsc.subcore_barrier()

        @pl.loop(0, nzc)
        def _(r):
            pltpu.sync_copy(acc.at[pl.ds(zb + r * zch, zch)],
                            s_hbm.at[pl.ds(cid * n + zb + r * zch, zch)])

    return agg_kernel(rows2, col, g)


_BN = 400


def _tc_prolog(x, degp, n):
    din = x.shape[1]
    h = din // 2
    nb = n // _BN

    def body(x_ref, d0, d1, g_ref, dinv_ref):
        dinv = lax.rsqrt(d0[:, 0:1] + d1[:, 0:1] + 1.0)
        g = x_ref[...] * dinv
        g_ref[0] = g[:, :h]
        g_ref[1] = g[:, h:]
        dinv_ref[...] = dinv

    return pl.pallas_call(
        body,
        grid=(nb,),
        in_specs=[
            pl.BlockSpec((_BN, din), lambda i: (i, 0)),
            pl.BlockSpec((_BN, 16), lambda i: (i, 0)),
            pl.BlockSpec((_BN, 16), lambda i: (i + nb, 0)),
        ],
        out_specs=[
            pl.BlockSpec((2, _BN, h), lambda i: (0, i, 0)),
            pl.BlockSpec((_BN, 1), lambda i: (i, 0)),
        ],
        out_shape=[
            jax.ShapeDtypeStruct((2, n, h), _F32),
            jax.ShapeDtypeStruct((n, 1), _F32),
        ],
    )(x, degp, degp)


def _tc_layer(s, g, dinv, w, b, n, w2=None):
    hf = s.shape[1]
    dout = (w2 if w2 is not None else w).shape[1]
    ho = dout // 2
    nb = n // _BN

    def body(s0, s1, g0, g1, dv, w_ref, b_ref, *rest):
        if w2 is not None:
            w2_ref, out = rest
        else:
            (out,) = rest
        t = jnp.concatenate([s0[...] + g0[...], s1[...] + g1[...]], axis=1)
        t = t * dv[...]
        hh = jnp.dot(t, w_ref[...], precision=_HI,
                     preferred_element_type=_F32) + b_ref[...]
        hh = jnp.maximum(hh, 0.0)
        if w2 is not None:
            hh = jnp.dot(hh, w2_ref[...], precision=_HI,
                         preferred_element_type=_F32)
        gg = hh * dv[...]
        out[0] = gg[:, :ho]
        out[1] = gg[:, ho:]

    in_specs = [
        pl.BlockSpec((_BN, hf), lambda i: (i, 0)),
        pl.BlockSpec((_BN, hf), lambda i: (i + nb, 0)),
        pl.BlockSpec((_BN, hf), lambda i: (i, 0)),
        pl.BlockSpec((_BN, hf), lambda i: (i + nb, 0)),
        pl.BlockSpec((_BN, 1), lambda i: (i, 0)),
        pl.BlockSpec(w.shape, lambda i: (0, 0)),
        pl.BlockSpec(b.shape, lambda i: (0, 0)),
    ]
    args = [s, s, g, g, dinv, w, b]
    if w2 is not None:
        in_specs.append(pl.BlockSpec(w2.shape, lambda i: (0, 0)))
        args.append(w2)
    return pl.pallas_call(
        body,
        grid=(nb,),
        in_specs=in_specs,
        out_specs=pl.BlockSpec((2, _BN, ho), lambda i: (0, i, 0)),
        out_shape=jax.ShapeDtypeStruct((2, n, ho), _F32),
    )(*args)


def _tc_epilog(s, g, dinv, b34, n):
    hf = s.shape[1]
    nb = n // _BN

    def body(s0, s1, g0, g1, dv, b_ref, out):
        t = jnp.concatenate([s0[...] + g0[...], s1[...] + g1[...]], axis=1)
        out[...] = t * dv[...] + b_ref[...]

    return pl.pallas_call(
        body,
        grid=(nb,),
        in_specs=[
            pl.BlockSpec((_BN, hf), lambda i: (i, 0)),
            pl.BlockSpec((_BN, hf), lambda i: (i + nb, 0)),
            pl.BlockSpec((_BN, hf), lambda i: (i, 0)),
            pl.BlockSpec((_BN, hf), lambda i: (i + nb, 0)),
            pl.BlockSpec((_BN, 1), lambda i: (i, 0)),
            pl.BlockSpec(b34.shape, lambda i: (0, 0)),
        ],
        out_specs=pl.BlockSpec((_BN, 2 * hf), lambda i: (i, 0)),
        out_shape=jax.ShapeDtypeStruct((n, 2 * hf), _F32),
    )(s, s, g, g, dinv, b34)


def _tc_fold(w3, wout, b3, bout):
    def body(w3_ref, wo_ref, b3_ref, bo_ref, w34_ref, b34_ref):
        w34_ref[...] = jnp.dot(w3_ref[...], wo_ref[...], precision=_HI,
                               preferred_element_type=_F32)
        b34_ref[...] = jnp.dot(b3_ref[...], wo_ref[...], precision=_HI,
                               preferred_element_type=_F32) + bo_ref[...]

    return pl.pallas_call(
        body,
        out_shape=[
            jax.ShapeDtypeStruct((w3.shape[0], wout.shape[1]), _F32),
            jax.ShapeDtypeStruct((1, wout.shape[1]), _F32),
        ],
    )(w3, wout, b3, bout)


def kernel(x, edge_index, W1, b1, W2, b2, W3, b3, Wout, bout):
    n, din = x.shape
    e = edge_index.shape[1]
    assert e % CHUNK == 0
    row = edge_index[0]
    col = edge_index[1]
    rows2 = jnp.concatenate([row, row + jnp.int32(n)])

    b1r = b1.reshape(1, -1)
    b2r = b2.reshape(1, -1)
    b3r = b3.reshape(1, -1)
    boutr = bout.reshape(1, -1)

    w34, b34 = _tc_fold(W3, Wout, b3r, boutr)

    degp = _sc_deg(col, n)
    g1_3d, dinv = _tc_prolog(x, degp, n)
    g1 = g1_3d.reshape(2 * n, din // 2)
    s1 = _sc_agg(rows2, col, g1, n, din // 2)

    g2_3d = _tc_layer(s1, g1, dinv, W1, b1r, n)
    g2 = g2_3d.reshape(2 * n, W1.shape[1] // 2)
    s2 = _sc_agg(rows2, col, g2, n, W1.shape[1] // 2, nb=2, ck=CHUNK)

    g3_3d = _tc_layer(s2, g2, dinv, W2, b2r, n, w2=w34)
    g3 = g3_3d.reshape(2 * n, Wout.shape[1] // 2)
    s3 = _sc_agg(rows2, col, g3, n, Wout.shape[1] // 2)

    return _tc_epilog(s3, g3, dinv, b34, n)


# speedup vs baseline: 21.5489x; 1.0482x over previous
import functools
import math

import jax
import jax.numpy as jnp
from jax import lax
from jax.experimental import pallas as pl
from jax.experimental.pallas import tpu as pltpu
from jax.experimental.pallas import tpu_sc as plsc

NC = 2
NS = 16
LANES = 16
CHUNK = 128

_HI = lax.Precision.HIGHEST
_F32 = jnp.float32


def _row_split(n):
    for mult in (80, 40, 16, 8):
        span = -(-(-(-n // NS)) // mult) * mult
        tail = n - (NS - 1) * span
        if 0 < tail <= span and tail % 8 == 0:
            return span, tail, math.gcd(span, tail)
    raise ValueError(f"no 8-aligned row split for n={n}")


def _fill2d(ref, rows, width, val):
    @pl.loop(0, rows)
    def _(i):
        @pl.loop(0, width, step=LANES)
        def _(j):
            ref[i, pl.ds(j, LANES)] = jnp.full((LANES,), val, _F32)


NB = 4


def _chunk_split(nchunks, nworkers):
    base, extra = divmod(nchunks, nworkers)
    return base, extra


def _worker_chunks(wid, base, extra):
    start = wid * base + jnp.minimum(wid, extra)
    cnt = base + jnp.where(wid < extra, 1, 0)
    return start, cnt


def _sc_deg(col, n):
    ck = 2 * CHUNK
    nchunks = col.shape[0] // ck
    assert nchunks * ck == col.shape[0]
    span, tail, zch = _row_split(n)
    base, extra = _chunk_split(nchunks, NC * NS)

    @functools.partial(
        pl.kernel,
        out_type=jax.ShapeDtypeStruct((2 * n, 128), _F32),
        mesh=plsc.VectorSubcoreMesh(core_axis_name="c", subcore_axis_name="s"),
        scratch_types=[
            pltpu.VMEM((ck, 16), _F32),
            pltpu.VMEM((NB, ck), jnp.int32),
            pltpu.VMEM((zch, 16), _F32),
            pltpu.VMEM_SHARED((n, 16), _F32),
            pltpu.SemaphoreType.DMA((NB,)),
            pltpu.SemaphoreType.DMA((NB,)),
        ],
        compiler_params=pltpu.CompilerParams(use_tc_tiling_on_sc=False),
    )
    def deg_kernel(col_hbm, out_hbm, ones_v, cidx, zbuf, acc, sem_i, sem_s):
        cid = lax.axis_index("c")
        sid = lax.axis_index("s")
        _fill2d(ones_v, ck, 16, 1.0)
        _fill2d(zbuf, zch, 16, 0.0)
        zb = sid * span
        nzc = jnp.minimum(span, n - zb) // zch

        @pl.loop(0, nzc)
        def _(r):
            pltpu.sync_copy(zbuf, acc.at[pl.ds(zb + r * zch, zch)])

        plsc.subcore_barrier()
        cstart, ccnt = _worker_chunks(cid * NS + sid, base, extra)
        ngrp = ccnt // NB

        @pl.loop(0, ngrp)
        def _(g):
            cb = (cstart + g * NB) * ck
            ld = [pltpu.async_copy(col_hbm.at[pl.ds(cb + b * ck, ck)],
                                   cidx.at[b], sem_i.at[b]) for b in range(NB)]
            st = []
            for b in range(NB):
                ld[b].wait()
                st.append(pltpu.async_copy(ones_v, acc.at[cidx.at[b]],
                                           sem_s.at[b], add=True))
            for b in range(NB):
                st[b].wait()

        @pl.loop(ngrp * NB, ccnt)
        def _(k):
            pltpu.sync_copy(col_hbm.at[pl.ds((cstart + k) * ck, ck)],
                            cidx.at[0])
            pltpu.sync_copy(ones_v, acc.at[cidx.at[0]], add=True)

        plsc.subcore_barrier()

        @pl.loop(0, nzc)
        def _(r):
            pltpu.sync_copy(acc.at[pl.ds(zb + r * zch, zch)],
                            out_hbm.at[pl.ds(cid * n + zb + r * zch, zch),
                                       pl.ds(0, 16)])

    return deg_kernel(col)


def _sc_agg(rows2, col, g, n, hf, nb=NB, ck=2 * CHUNK):
    e = col.shape[0]
    nchunks = e // ck
    assert nchunks * ck == e and hf % LANES == 0
    span, tail, zch = _row_split(n)
    while zch * hf > 5120 and zch % 16 == 0:
        zch //= 2
    base, extra = _chunk_split(nchunks, NS)

    @functools.partial(
        pl.kernel,
        out_type=jax.ShapeDtypeStruct((2 * n, hf), _F32),
        mesh=plsc.VectorSubcoreMesh(core_axis_name="c", subcore_axis_name="s"),
        scratch_types=[
            pltpu.VMEM((nb, ck), jnp.int32),
            pltpu.VMEM((nb, ck), jnp.int32),
            pltpu.VMEM((nb, ck, hf), _F32),
            pltpu.VMEM((zch, hf), _F32),
            pltpu.VMEM_SHARED((n, hf), _F32),
            pltpu.SemaphoreType.DMA((nb,)),
            pltpu.SemaphoreType.DMA((nb,)),
            pltpu.SemaphoreType.DMA((nb,)),
            pltpu.SemaphoreType.DMA((nb,)),
        ],
        compiler_params=pltpu.CompilerParams(use_tc_tiling_on_sc=False),
    )
    def agg_kernel(rows_hbm, col_hbm, g_hbm, s_hbm,
                   ridx, cidx, gbuf, zbuf, acc, sem_ir, sem_ic, sem_g, sem_s):
        cid = lax.axis_index("c")
        sid = lax.axis_index("s")
        _fill2d(zbuf, zch, hf, 0.0)
        zb = sid * span
        nzc = jnp.minimum(span, n - zb) // zch

        @pl.loop(0, nzc)
        def _(r):
            pltpu.sync_copy(zbuf, acc.at[pl.ds(zb + r * zch, zch)])

        plsc.subcore_barrier()
        cstart, ccnt = _worker_chunks(sid, base, extra)
        ngrp = ccnt // nb

        @pl.loop(0, ngrp)
        def _(grp):
            cb = (cstart + grp * nb) * ck
            ldr = [pltpu.async_copy(
                rows_hbm.at[pl.ds(cid * e + cb + b * ck, ck)],
                ridx.at[b], sem_ir.at[b]) for b in range(nb)]
            ldc = [pltpu.async_copy(
                col_hbm.at[pl.ds(cb + b * ck, ck)],
                cidx.at[b], sem_ic.at[b]) for b in range(nb)]
            gth = []
            for b in range(nb):
                ldr[b].wait()
                gth.append(pltpu.async_copy(g_hbm.at[ridx.at[b]], gbuf.at[b],
                                            sem_g.at[b]))
            sct = []
            for b in range(nb):
                gth[b].wait()
                ldc[b].wait()
                sct.append(pltpu.async_copy(gbuf.at[b], acc.at[cidx.at[b]],
                                            sem_s.at[b], add=True))
            for b in range(nb):
                sct[b].wait()

        @pl.loop(ngrp * nb, ccnt)
        def _(k):
            kb = (cstart + k) * ck
            pltpu.sync_copy(rows_hbm.at[pl.ds(cid * e + kb, ck)],
                            ridx.at[0])
            pltpu.sync_copy(col_hbm.at[pl.ds(kb, ck)], cidx.at[0])
            pltpu.sync_copy(g_hbm.at[ridx.at[0]], gbuf.at[0])
            pltpu.sync_copy(gbuf.at[0], acc.at[cidx.at[0]], add=True)

        plsc.subcore_barrier()

        @pl.loop(0, nzc)
        def _(r):
            pltpu.sync_copy(acc.at[pl.ds(zb + r * zch, zch)],
                            s_hbm.at[pl.ds(cid * n + zb + r * zch, zch)])

    return agg_kernel(rows2, col, g)


def _sc_agg_es(row, col, g, n):
    e = col.shape[0]
    ck = CHUNK
    nchunks = e // ck
    assert nchunks * ck == e
    span, tail, zch = _row_split(n)
    while zch * 128 > 5120 and zch % 16 == 0:
        zch //= 2
    base, extra = _chunk_split(nchunks, NC * NS)
    nb = 2

    @functools.partial(
        pl.kernel,
        out_type=jax.ShapeDtypeStruct((2 * n, 128), _F32),
        mesh=plsc.VectorSubcoreMesh(core_axis_name="c", subcore_axis_name="s"),
        scratch_types=[
            pltpu.VMEM((nb, ck), jnp.int32),
            pltpu.VMEM((nb, ck), jnp.int32),
            pltpu.VMEM((nb, ck, 128), _F32),
            pltpu.VMEM((zch, 128), _F32),
            pltpu.VMEM_SHARED((n, 128), _F32),
            pltpu.SemaphoreType.DMA((nb,)),
            pltpu.SemaphoreType.DMA((nb,)),
            pltpu.SemaphoreType.DMA((nb,)),
            pltpu.SemaphoreType.DMA((nb,)),
        ],
        compiler_params=pltpu.CompilerParams(use_tc_tiling_on_sc=False),
    )
    def agg_kernel(row_hbm, col_hbm, g_hbm, s_hbm,
                   ridx, cidx, gbuf, zbuf, acc, sem_ir, sem_ic, sem_g, sem_s):
        cid = lax.axis_index("c")
        sid = lax.axis_index("s")
        _fill2d(zbuf, zch, 128, 0.0)
        zb = sid * span
        nzc = jnp.minimum(span, n - zb) // zch

        @pl.loop(0, nzc)
        def _(r):
            pltpu.sync_copy(zbuf, acc.at[pl.ds(zb + r * zch, zch)])

        plsc.subcore_barrier()
        cstart, ccnt = _worker_chunks(cid * NS + sid, base, extra)
        ngrp = ccnt // nb

        @pl.loop(0, ngrp)
        def _(grp):
            cb = (cstart + grp * nb) * ck
            ldr = [pltpu.async_copy(row_hbm.at[pl.ds(cb + b * ck, ck)],
                                    ridx.at[b], sem_ir.at[b])
                   for b in range(nb)]
            ldc = [pltpu.async_copy(col_hbm.at[pl.ds(cb + b * ck, ck)],
                                    cidx.at[b], sem_ic.at[b])
                   for b in range(nb)]
            gth = []
            for b in range(nb):
                ldr[b].wait()
                gth.append(pltpu.async_copy(g_hbm.at[ridx.at[b]], gbuf.at[b],
                                            sem_g.at[b]))
            sct = []
            for b in range(nb):
                gth[b].wait()
                ldc[b].wait()
                sct.append(pltpu.async_copy(gbuf.at[b], acc.at[cidx.at[b]],
                                            sem_s.at[b], add=True))
            for b in range(nb):
                sct[b].wait()

        @pl.loop(ngrp * nb, ccnt)
        def _(k):
            kb = (cstart + k) * ck
            pltpu.sync_copy(row_hbm.at[pl.ds(kb, ck)], ridx.at[0])
            pltpu.sync_copy(col_hbm.at[pl.ds(kb, ck)], cidx.at[0])
            pltpu.sync_copy(g_hbm.at[ridx.at[0]], gbuf.at[0])
            pltpu.sync_copy(gbuf.at[0], acc.at[cidx.at[0]], add=True)

        plsc.subcore_barrier()

        @pl.loop(0, nzc)
        def _(r):
            pltpu.sync_copy(acc.at[pl.ds(zb + r * zch, zch)],
                            s_hbm.at[pl.ds(cid * n + zb + r * zch, zch)])

    return agg_kernel(row, col, g)


_BN = 1000


def _tc_prolog(x, degp, n):
    din = x.shape[1]
    nb = n // _BN

    def body(x_ref, d0, d1, g_ref, dinv_ref):
        dinv = lax.rsqrt(d0[:, 0:1] + d1[:, 0:1] + 1.0)
        g_ref[...] = x_ref[...] * dinv
        dinv_ref[...] = dinv

    return pl.pallas_call(
        body,
        grid=(nb,),
        in_specs=[
            pl.BlockSpec((_BN, din), lambda i: (i, 0)),
            pl.BlockSpec((_BN, 128), lambda i: (i, 0)),
            pl.BlockSpec((_BN, 128), lambda i: (i + nb, 0)),
        ],
        out_specs=[
            pl.BlockSpec((_BN, din), lambda i: (i, 0)),
            pl.BlockSpec((_BN, 1), lambda i: (i, 0)),
        ],
        out_shape=[
            jax.ShapeDtypeStruct((n, din), _F32),
            jax.ShapeDtypeStruct((n, 1), _F32),
        ],
    )(x, degp, degp)


def _tc_layer1(s, g, dinv, w, b, n):
    din = g.shape[1]
    dout = w.shape[1]
    ho = dout // 2
    nb = n // _BN

    def body(p0, p1, g0, dv, w_ref, b_ref, out):
        t = (p0[...] + p1[...] + g0[...]) * dv[...]
        hh = jnp.dot(t, w_ref[...], precision=_HI,
                     preferred_element_type=_F32) + b_ref[...]
        hh = jnp.maximum(hh, 0.0)
        gg = hh * dv[...]
        out[0] = gg[:, :ho]
        out[1] = gg[:, ho:]

    return pl.pallas_call(
        body,
        grid=(nb,),
        in_specs=[
            pl.BlockSpec((_BN, din), lambda i: (i, 0)),
            pl.BlockSpec((_BN, din), lambda i: (i + nb, 0)),
            pl.BlockSpec((_BN, din), lambda i: (i, 0)),
            pl.BlockSpec((_BN, 1), lambda i: (i, 0)),
            pl.BlockSpec(w.shape, lambda i: (0, 0)),
            pl.BlockSpec(b.shape, lambda i: (0, 0)),
        ],
        out_specs=pl.BlockSpec((2, _BN, ho), lambda i: (0, i, 0)),
        out_shape=jax.ShapeDtypeStruct((2, n, ho), _F32),
    )(s, s, g, dinv, w, b)


def _tc_layer(s, g, dinv, w, b, n, w2=None):
    hf = s.shape[1]
    dout = (w2 if w2 is not None else w).shape[1]
    ho = dout // 2
    nb = n // _BN

    def body(s0, s1, g0, g1, dv, w_ref, b_ref, *rest):
        if w2 is not None:
            w2_ref, out = rest
        else:
            (out,) = rest
        t = jnp.concatenate([s0[...] + g0[...], s1[...] + g1[...]], axis=1)
        t = t * dv[...]
        hh = jnp.dot(t, w_ref[...], precision=_HI,
                     preferred_element_type=_F32) + b_ref[...]
        hh = jnp.maximum(hh, 0.0)
        if w2 is not None:
            hh = jnp.dot(hh, w2_ref[...], precision=_HI,
                         preferred_element_type=_F32)
        gg = hh * dv[...]
        out[0] = gg[:, :ho]
        out[1] = gg[:, ho:]

    in_specs = [
        pl.BlockSpec((_BN, hf), lambda i: (i, 0)),
        pl.BlockSpec((_BN, hf), lambda i: (i + nb, 0)),
        pl.BlockSpec((_BN, hf), lambda i: (i, 0)),
        pl.BlockSpec((_BN, hf), lambda i: (i + nb, 0)),
        pl.BlockSpec((_BN, 1), lambda i: (i, 0)),
        pl.BlockSpec(w.shape, lambda i: (0, 0)),
        pl.BlockSpec(b.shape, lambda i: (0, 0)),
    ]
    args = [s, s, g, g, dinv, w, b]
    if w2 is not None:
        in_specs.append(pl.BlockSpec(w2.shape, lambda i: (0, 0)))
        args.append(w2)
    return pl.pallas_call(
        body,
        grid=(nb,),
        in_specs=in_specs,
        out_specs=pl.BlockSpec((2, _BN, ho), lambda i: (0, i, 0)),
        out_shape=jax.ShapeDtypeStruct((2, n, ho), _F32),
    )(*args)


def _tc_epilog(s, g, dinv, b34, n):
    hf = s.shape[1]
    nb = n // _BN

    def body(s0, s1, g0, g1, dv, b_ref, out):
        t = jnp.concatenate([s0[...] + g0[...], s1[...] + g1[...]], axis=1)
        out[...] = t * dv[...] + b_ref[...]

    return pl.pallas_call(
        body,
        grid=(nb,),
        in_specs=[
            pl.BlockSpec((_BN, hf), lambda i: (i, 0)),
            pl.BlockSpec((_BN, hf), lambda i: (i + nb, 0)),
            pl.BlockSpec((_BN, hf), lambda i: (i, 0)),
            pl.BlockSpec((_BN, hf), lambda i: (i + nb, 0)),
            pl.BlockSpec((_BN, 1), lambda i: (i, 0)),
            pl.BlockSpec(b34.shape, lambda i: (0, 0)),
        ],
        out_specs=pl.BlockSpec((_BN, 2 * hf), lambda i: (i, 0)),
        out_shape=jax.ShapeDtypeStruct((n, 2 * hf), _F32),
    )(s, s, g, g, dinv, b34)


def _tc_fold(w3, wout, b3, bout):
    def body(w3_ref, wo_ref, b3_ref, bo_ref, w34_ref, b34_ref):
        w34_ref[...] = jnp.dot(w3_ref[...], wo_ref[...], precision=_HI,
                               preferred_element_type=_F32)
        b34_ref[...] = jnp.dot(b3_ref[...], wo_ref[...], precision=_HI,
                               preferred_element_type=_F32) + bo_ref[...]

    return pl.pallas_call(
        body,
        out_shape=[
            jax.ShapeDtypeStruct((w3.shape[0], wout.shape[1]), _F32),
            jax.ShapeDtypeStruct((1, wout.shape[1]), _F32),
        ],
    )(w3, wout, b3, bout)


def kernel(x, edge_index, W1, b1, W2, b2, W3, b3, Wout, bout):
    n, din = x.shape
    e = edge_index.shape[1]
    assert e % CHUNK == 0
    row = edge_index[0]
    col = edge_index[1]
    rows2 = jnp.concatenate([row, row + jnp.int32(n)])

    b1r = b1.reshape(1, -1)
    b2r = b2.reshape(1, -1)
    b3r = b3.reshape(1, -1)
    boutr = bout.reshape(1, -1)

    w34, b34 = _tc_fold(W3, Wout, b3r, boutr)

    degp = _sc_deg(col, n)
    g1, dinv = _tc_prolog(x, degp, n)
    s1 = _sc_agg_es(row, col, g1, n)

    g2_3d = _tc_layer1(s1, g1, dinv, W1, b1r, n)
    g2 = g2_3d.reshape(2 * n, W1.shape[1] // 2)
    s2 = _sc_agg(rows2, col, g2, n, W1.shape[1] // 2, nb=2, ck=CHUNK)

    g3_3d = _tc_layer(s2, g2, dinv, W2, b2r, n, w2=w34)
    g3 = g3_3d.reshape(2 * n, Wout.shape[1] // 2)
    s3 = _sc_agg(rows2, col, g3, n, Wout.shape[1] // 2)

    return _tc_epilog(s3, g3, dinv, b34, n)


# speedup vs baseline: 21.8669x; 1.0148x over previous
import functools
import math

import jax
import jax.numpy as jnp
from jax import lax
from jax.experimental import pallas as pl
from jax.experimental.pallas import tpu as pltpu
from jax.experimental.pallas import tpu_sc as plsc

NC = 2
NS = 16
LANES = 16
CHUNK = 128

_HI = lax.Precision.HIGHEST
_F32 = jnp.float32


def _row_split(n):
    for mult in (80, 40, 16, 8):
        span = -(-(-(-n // NS)) // mult) * mult
        tail = n - (NS - 1) * span
        if 0 < tail <= span and tail % 8 == 0:
            return span, tail, math.gcd(span, tail)
    raise ValueError(f"no 8-aligned row split for n={n}")


def _fill2d(ref, rows, width, val):
    @pl.loop(0, rows)
    def _(i):
        @pl.loop(0, width, step=LANES)
        def _(j):
            ref[i, pl.ds(j, LANES)] = jnp.full((LANES,), val, _F32)


NB = 4


def _chunk_split(nchunks, nworkers):
    base, extra = divmod(nchunks, nworkers)
    return base, extra


def _worker_chunks(wid, base, extra):
    start = wid * base + jnp.minimum(wid, extra)
    cnt = base + jnp.where(wid < extra, 1, 0)
    return start, cnt


def _sc_deg(eidx, n):
    e = eidx.shape[0] // 2
    ck = 2 * CHUNK
    nchunks = e // ck
    assert nchunks * ck == e
    span, tail, zch = _row_split(n)
    base, extra = _chunk_split(nchunks, NC * NS)

    @functools.partial(
        pl.kernel,
        out_type=jax.ShapeDtypeStruct((2 * n, 128), _F32),
        mesh=plsc.VectorSubcoreMesh(core_axis_name="c", subcore_axis_name="s"),
        scratch_types=[
            pltpu.VMEM((ck, 16), _F32),
            pltpu.VMEM((NB, ck), jnp.int32),
            pltpu.VMEM((zch, 16), _F32),
            pltpu.VMEM_SHARED((n, 16), _F32),
            pltpu.SemaphoreType.DMA((NB,)),
            pltpu.SemaphoreType.DMA((NB,)),
        ],
        compiler_params=pltpu.CompilerParams(use_tc_tiling_on_sc=False),
    )
    def deg_kernel(col_hbm, out_hbm, ones_v, cidx, zbuf, acc, sem_i, sem_s):
        cid = lax.axis_index("c")
        sid = lax.axis_index("s")
        _fill2d(ones_v, ck, 16, 1.0)
        _fill2d(zbuf, zch, 16, 0.0)
        zb = sid * span
        nzc = jnp.minimum(span, n - zb) // zch

        @pl.loop(0, nzc)
        def _(r):
            pltpu.sync_copy(zbuf, acc.at[pl.ds(zb + r * zch, zch)])

        plsc.subcore_barrier()
        cstart, ccnt = _worker_chunks(cid * NS + sid, base, extra)
        ngrp = ccnt // NB

        @pl.loop(0, ngrp)
        def _(g):
            cb = e + (cstart + g * NB) * ck
            ld = [pltpu.async_copy(col_hbm.at[pl.ds(cb + b * ck, ck)],
                                   cidx.at[b], sem_i.at[b]) for b in range(NB)]
            st = []
            for b in range(NB):
                ld[b].wait()
                st.append(pltpu.async_copy(ones_v, acc.at[cidx.at[b]],
                                           sem_s.at[b], add=True))
            for b in range(NB):
                st[b].wait()

        @pl.loop(ngrp * NB, ccnt)
        def _(k):
            pltpu.sync_copy(col_hbm.at[pl.ds(e + (cstart + k) * ck, ck)],
                            cidx.at[0])
            pltpu.sync_copy(ones_v, acc.at[cidx.at[0]], add=True)

        plsc.subcore_barrier()

        @pl.loop(0, nzc)
        def _(r):
            pltpu.sync_copy(acc.at[pl.ds(zb + r * zch, zch)],
                            out_hbm.at[pl.ds(cid * n + zb + r * zch, zch),
                                       pl.ds(0, 16)])

    return deg_kernel(eidx)


def _sc_agg(eidx, g, n, hf, nb=NB, ck=2 * CHUNK):
    e = eidx.shape[0] // 2
    nchunks = e // ck
    assert nchunks * ck == e and hf % LANES == 0
    span, tail, zch = _row_split(n)
    while zch * hf > 5120 and zch % 16 == 0:
        zch //= 2
    base, extra = _chunk_split(nchunks, NS)

    @functools.partial(
        pl.kernel,
        out_type=jax.ShapeDtypeStruct((2 * n, hf), _F32),
        mesh=plsc.VectorSubcoreMesh(core_axis_name="c", subcore_axis_name="s"),
        scratch_types=[
            pltpu.VMEM((nb, ck), jnp.int32),
            pltpu.VMEM((nb, ck), jnp.int32),
            pltpu.VMEM((nb, ck, hf), _F32),
            pltpu.VMEM((zch, hf), _F32),
            pltpu.VMEM_SHARED((n, hf), _F32),
            pltpu.SemaphoreType.DMA((nb,)),
            pltpu.SemaphoreType.DMA((nb,)),
            pltpu.SemaphoreType.DMA((nb,)),
            pltpu.SemaphoreType.DMA((nb,)),
        ],
        compiler_params=pltpu.CompilerParams(use_tc_tiling_on_sc=False),
    )
    def agg_kernel(eidx_hbm, g_hbm, s_hbm,
                   ridx, cidx, gbuf, zbuf, acc, sem_ir, sem_ic, sem_g, sem_s):
        cid = lax.axis_index("c")
        sid = lax.axis_index("s")
        gwin = g_hbm.at[pl.ds(cid * n, n)]
        _fill2d(zbuf, zch, hf, 0.0)
        zb = sid * span
        nzc = jnp.minimum(span, n - zb) // zch

        @pl.loop(0, nzc)
        def _(r):
            pltpu.sync_copy(zbuf, acc.at[pl.ds(zb + r * zch, zch)])

        plsc.subcore_barrier()
        cstart, ccnt = _worker_chunks(sid, base, extra)
        ngrp = ccnt // nb

        @pl.loop(0, ngrp)
        def _(grp):
            cb = (cstart + grp * nb) * ck
            ldr = [pltpu.async_copy(
                eidx_hbm.at[pl.ds(cb + b * ck, ck)],
                ridx.at[b], sem_ir.at[b]) for b in range(nb)]
            ldc = [pltpu.async_copy(
                eidx_hbm.at[pl.ds(e + cb + b * ck, ck)],
                cidx.at[b], sem_ic.at[b]) for b in range(nb)]
            gth = []
            for b in range(nb):
                ldr[b].wait()
                gth.append(pltpu.async_copy(gwin.at[ridx.at[b]], gbuf.at[b],
                                            sem_g.at[b]))
            sct = []
            for b in range(nb):
                gth[b].wait()
                ldc[b].wait()
                sct.append(pltpu.async_copy(gbuf.at[b], acc.at[cidx.at[b]],
                                            sem_s.at[b], add=True))
            for b in range(nb):
                sct[b].wait()

        @pl.loop(ngrp * nb, ccnt)
        def _(k):
            kb = (cstart + k) * ck
            pltpu.sync_copy(eidx_hbm.at[pl.ds(kb, ck)], ridx.at[0])
            pltpu.sync_copy(eidx_hbm.at[pl.ds(e + kb, ck)], cidx.at[0])
            pltpu.sync_copy(gwin.at[ridx.at[0]], gbuf.at[0])
            pltpu.sync_copy(gbuf.at[0], acc.at[cidx.at[0]], add=True)

        plsc.subcore_barrier()

        @pl.loop(0, nzc)
        def _(r):
            pltpu.sync_copy(acc.at[pl.ds(zb + r * zch, zch)],
                            s_hbm.at[pl.ds(cid * n + zb + r * zch, zch)])

    return agg_kernel(eidx, g)


def _sc_agg_es(eidx, g, n):
    e = eidx.shape[0] // 2
    ck = CHUNK
    nchunks = e // ck
    assert nchunks * ck == e
    span, tail, zch = _row_split(n)
    while zch * 128 > 5120 and zch % 16 == 0:
        zch //= 2
    base, extra = _chunk_split(nchunks, NC * NS)
    nb = 2

    @functools.partial(
        pl.kernel,
        out_type=jax.ShapeDtypeStruct((2 * n, 128), _F32),
        mesh=plsc.VectorSubcoreMesh(core_axis_name="c", subcore_axis_name="s"),
        scratch_types=[
            pltpu.VMEM((nb, ck), jnp.int32),
            pltpu.VMEM((nb, ck), jnp.int32),
            pltpu.VMEM((nb, ck, 128), _F32),
            pltpu.VMEM((zch, 128), _F32),
            pltpu.VMEM_SHARED((n, 128), _F32),
            pltpu.SemaphoreType.DMA((nb,)),
            pltpu.SemaphoreType.DMA((nb,)),
            pltpu.SemaphoreType.DMA((nb,)),
            pltpu.SemaphoreType.DMA((nb,)),
        ],
        compiler_params=pltpu.CompilerParams(use_tc_tiling_on_sc=False),
    )
    def agg_kernel(eidx_hbm, g_hbm, s_hbm,
                   ridx, cidx, gbuf, zbuf, acc, sem_ir, sem_ic, sem_g, sem_s):
        cid = lax.axis_index("c")
        sid = lax.axis_index("s")
        _fill2d(zbuf, zch, 128, 0.0)
        zb = sid * span
        nzc = jnp.minimum(span, n - zb) // zch

        @pl.loop(0, nzc)
        def _(r):
            pltpu.sync_copy(zbuf, acc.at[pl.ds(zb + r * zch, zch)])

        plsc.subcore_barrier()
        cstart, ccnt = _worker_chunks(cid * NS + sid, base, extra)
        ngrp = ccnt // nb

        @pl.loop(0, ngrp)
        def _(grp):
            cb = (cstart + grp * nb) * ck
            ldr = [pltpu.async_copy(eidx_hbm.at[pl.ds(cb + b * ck, ck)],
                                    ridx.at[b], sem_ir.at[b])
                   for b in range(nb)]
            ldc = [pltpu.async_copy(eidx_hbm.at[pl.ds(e + cb + b * ck, ck)],
                                    cidx.at[b], sem_ic.at[b])
                   for b in range(nb)]
            gth = []
            for b in range(nb):
                ldr[b].wait()
                gth.append(pltpu.async_copy(g_hbm.at[ridx.at[b]], gbuf.at[b],
                                            sem_g.at[b]))
            sct = []
            for b in range(nb):
                gth[b].wait()
                ldc[b].wait()
                sct.append(pltpu.async_copy(gbuf.at[b], acc.at[cidx.at[b]],
                                            sem_s.at[b], add=True))
            for b in range(nb):
                sct[b].wait()

        @pl.loop(ngrp * nb, ccnt)
        def _(k):
            kb = (cstart + k) * ck
            pltpu.sync_copy(eidx_hbm.at[pl.ds(kb, ck)], ridx.at[0])
            pltpu.sync_copy(eidx_hbm.at[pl.ds(e + kb, ck)], cidx.at[0])
            pltpu.sync_copy(g_hbm.at[ridx.at[0]], gbuf.at[0])
            pltpu.sync_copy(gbuf.at[0], acc.at[cidx.at[0]], add=True)

        plsc.subcore_barrier()

        @pl.loop(0, nzc)
        def _(r):
            pltpu.sync_copy(acc.at[pl.ds(zb + r * zch, zch)],
                            s_hbm.at[pl.ds(cid * n + zb + r * zch, zch)])

    return agg_kernel(eidx, g)


_BN = 1000


def _tc_prolog(x, degp, n):
    din = x.shape[1]
    nb = n // _BN

    def body(x_ref, d0, d1, g_ref, dinv_ref):
        dinv = lax.rsqrt(d0[:, 0:1] + d1[:, 0:1] + 1.0)
        g_ref[...] = x_ref[...] * dinv
        dinv_ref[...] = dinv

    return pl.pallas_call(
        body,
        grid=(nb,),
        in_specs=[
            pl.BlockSpec((_BN, din), lambda i: (i, 0)),
            pl.BlockSpec((_BN, 128), lambda i: (i, 0)),
            pl.BlockSpec((_BN, 128), lambda i: (i + nb, 0)),
        ],
        out_specs=[
            pl.BlockSpec((_BN, din), lambda i: (i, 0)),
            pl.BlockSpec((_BN, 1), lambda i: (i, 0)),
        ],
        out_shape=[
            jax.ShapeDtypeStruct((n, din), _F32),
            jax.ShapeDtypeStruct((n, 1), _F32),
        ],
    )(x, degp, degp)


def _tc_layer1(s, g, dinv, w, b, n):
    din = g.shape[1]
    dout = w.shape[1]
    ho = dout // 2
    nb = n // _BN

    def body(p0, p1, g0, dv, w_ref, b_ref, out):
        t = (p0[...] + p1[...] + g0[...]) * dv[...]
        hh = jnp.dot(t, w_ref[...], precision=_HI,
                     preferred_element_type=_F32) + b_ref[...]
        hh = jnp.maximum(hh, 0.0)
        gg = hh * dv[...]
        out[0] = gg[:, :ho]
        out[1] = gg[:, ho:]

    return pl.pallas_call(
        body,
        grid=(nb,),
        in_specs=[
            pl.BlockSpec((_BN, din), lambda i: (i, 0)),
            pl.BlockSpec((_BN, din), lambda i: (i + nb, 0)),
            pl.BlockSpec((_BN, din), lambda i: (i, 0)),
            pl.BlockSpec((_BN, 1), lambda i: (i, 0)),
            pl.BlockSpec(w.shape, lambda i: (0, 0)),
            pl.BlockSpec(b.shape, lambda i: (0, 0)),
        ],
        out_specs=pl.BlockSpec((2, _BN, ho), lambda i: (0, i, 0)),
        out_shape=jax.ShapeDtypeStruct((2, n, ho), _F32),
    )(s, s, g, dinv, w, b)


def _tc_layer(s, g, dinv, w, b, n, w2=None):
    hf = s.shape[1]
    dout = (w2 if w2 is not None else w).shape[1]
    ho = dout // 2
    nb = n // _BN

    def body(s0, s1, g0, g1, dv, w_ref, b_ref, *rest):
        if w2 is not None:
            w2_ref, out = rest
        else:
            (out,) = rest
        t = jnp.concatenate([s0[...] + g0[...], s1[...] + g1[...]], axis=1)
        t = t * dv[...]
        hh = jnp.dot(t, w_ref[...], precision=_HI,
                     preferred_element_type=_F32) + b_ref[...]
        hh = jnp.maximum(hh, 0.0)
        if w2 is not None:
            hh = jnp.dot(hh, w2_ref[...], precision=_HI,
                         preferred_element_type=_F32)
        gg = hh * dv[...]
        out[0] = gg[:, :ho]
        out[1] = gg[:, ho:]

    in_specs = [
        pl.BlockSpec((_BN, hf), lambda i: (i, 0)),
        pl.BlockSpec((_BN, hf), lambda i: (i + nb, 0)),
        pl.BlockSpec((_BN, hf), lambda i: (i, 0)),
        pl.BlockSpec((_BN, hf), lambda i: (i + nb, 0)),
        pl.BlockSpec((_BN, 1), lambda i: (i, 0)),
        pl.BlockSpec(w.shape, lambda i: (0, 0)),
        pl.BlockSpec(b.shape, lambda i: (0, 0)),
    ]
    args = [s, s, g, g, dinv, w, b]
    if w2 is not None:
        in_specs.append(pl.BlockSpec(w2.shape, lambda i: (0, 0)))
        args.append(w2)
    return pl.pallas_call(
        body,
        grid=(nb,),
        in_specs=in_specs,
        out_specs=pl.BlockSpec((2, _BN, ho), lambda i: (0, i, 0)),
        out_shape=jax.ShapeDtypeStruct((2, n, ho), _F32),
    )(*args)


def _tc_epilog(s, g, dinv, b34, n):
    hf = s.shape[1]
    nb = n // _BN

    def body(s0, s1, g0, g1, dv, b_ref, out):
        t = jnp.concatenate([s0[...] + g0[...], s1[...] + g1[...]], axis=1)
        out[...] = t * dv[...] + b_ref[...]

    return pl.pallas_call(
        body,
        grid=(nb,),
        in_specs=[
            pl.BlockSpec((_BN, hf), lambda i: (i, 0)),
            pl.BlockSpec((_BN, hf), lambda i: (i + nb, 0)),
            pl.BlockSpec((_BN, hf), lambda i: (i, 0)),
            pl.BlockSpec((_BN, hf), lambda i: (i + nb, 0)),
            pl.BlockSpec((_BN, 1), lambda i: (i, 0)),
            pl.BlockSpec(b34.shape, lambda i: (0, 0)),
        ],
        out_specs=pl.BlockSpec((_BN, 2 * hf), lambda i: (i, 0)),
        out_shape=jax.ShapeDtypeStruct((n, 2 * hf), _F32),
    )(s, s, g, g, dinv, b34)


def _tc_fold(w3, wout, b3, bout):
    def body(w3_ref, wo_ref, b3_ref, bo_ref, w34_ref, b34_ref):
        w34_ref[...] = jnp.dot(w3_ref[...], wo_ref[...], precision=_HI,
                               preferred_element_type=_F32)
        b34_ref[...] = jnp.dot(b3_ref[...], wo_ref[...], precision=_HI,
                               preferred_element_type=_F32) + bo_ref[...]

    return pl.pallas_call(
        body,
        out_shape=[
            jax.ShapeDtypeStruct((w3.shape[0], wout.shape[1]), _F32),
            jax.ShapeDtypeStruct((1, wout.shape[1]), _F32),
        ],
    )(w3, wout, b3, bout)


def kernel(x, edge_index, W1, b1, W2, b2, W3, b3, Wout, bout):
    n, din = x.shape
    e = edge_index.shape[1]
    assert e % (2 * CHUNK) == 0
    eidx = edge_index.reshape(2 * e)

    b1r = b1.reshape(1, -1)
    b2r = b2.reshape(1, -1)
    b3r = b3.reshape(1, -1)
    boutr = bout.reshape(1, -1)

    w34, b34 = _tc_fold(W3, Wout, b3r, boutr)

    degp = _sc_deg(eidx, n)
    g1, dinv = _tc_prolog(x, degp, n)
    s1 = _sc_agg_es(eidx, g1, n)

    g2_3d = _tc_layer1(s1, g1, dinv, W1, b1r, n)
    g2 = g2_3d.reshape(2 * n, W1.shape[1] // 2)
    s2 = _sc_agg(eidx, g2, n, W1.shape[1] // 2, nb=2, ck=CHUNK)

    g3_3d = _tc_layer(s2, g2, dinv, W2, b2r, n, w2=w34)
    g3 = g3_3d.reshape(2 * n, Wout.shape[1] // 2)
    s3 = _sc_agg(eidx, g3, n, Wout.shape[1] // 2)

    return _tc_epilog(s3, g3, dinv, b34, n)


# speedup vs baseline: 23.1148x; 1.0571x over previous
import functools
import math

import jax
import jax.numpy as jnp
from jax import lax
from jax.experimental import pallas as pl
from jax.experimental.pallas import tpu as pltpu
from jax.experimental.pallas import tpu_sc as plsc

NC = 2
NS = 16
LANES = 16
CHUNK = 128

_HI = lax.Precision.HIGHEST
_F32 = jnp.float32


def _row_split(n):
    for mult in (80, 40, 16, 8):
        span = -(-(-(-n // NS)) // mult) * mult
        tail = n - (NS - 1) * span
        if 0 < tail <= span and tail % 8 == 0:
            return span, tail, math.gcd(span, tail)
    raise ValueError(f"no 8-aligned row split for n={n}")


def _fill2d(ref, rows, width, val):
    @pl.loop(0, rows)
    def _(i):
        @pl.loop(0, width, step=LANES)
        def _(j):
            ref[i, pl.ds(j, LANES)] = jnp.full((LANES,), val, _F32)


NB = 4


def _chunk_split(nchunks, nworkers):
    base, extra = divmod(nchunks, nworkers)
    return base, extra


def _worker_chunks(wid, base, extra):
    start = wid * base + jnp.minimum(wid, extra)
    cnt = base + jnp.where(wid < extra, 1, 0)
    return start, cnt


def _sc_deg(eidx, n):
    e = eidx.shape[0] // 2
    ck = 2 * CHUNK
    nchunks = e // ck
    assert nchunks * ck == e
    span, tail, zch = _row_split(n)
    base, extra = _chunk_split(nchunks, NC * NS)

    @functools.partial(
        pl.kernel,
        out_type=jax.ShapeDtypeStruct((2 * n, 128), _F32),
        mesh=plsc.VectorSubcoreMesh(core_axis_name="c", subcore_axis_name="s"),
        scratch_types=[
            pltpu.VMEM((ck, 16), _F32),
            pltpu.VMEM((NB, ck), jnp.int32),
            pltpu.VMEM((zch, 16), _F32),
            pltpu.VMEM_SHARED((n, 16), _F32),
            pltpu.SemaphoreType.DMA((NB,)),
            pltpu.SemaphoreType.DMA((NB,)),
        ],
        compiler_params=pltpu.CompilerParams(use_tc_tiling_on_sc=False),
    )
    def deg_kernel(col_hbm, out_hbm, ones_v, cidx, zbuf, acc, sem_i, sem_s):
        cid = lax.axis_index("c")
        sid = lax.axis_index("s")
        _fill2d(ones_v, ck, 16, 1.0)
        _fill2d(zbuf, zch, 16, 0.0)
        zb = sid * span
        nzc = jnp.minimum(span, n - zb) // zch

        @pl.loop(0, nzc)
        def _(r):
            pltpu.sync_copy(zbuf, acc.at[pl.ds(zb + r * zch, zch)])

        plsc.subcore_barrier()
        cstart, ccnt = _worker_chunks(cid * NS + sid, base, extra)
        ngrp = ccnt // NB

        @pl.loop(0, ngrp)
        def _(g):
            cb = e + (cstart + g * NB) * ck
            ld = [pltpu.async_copy(col_hbm.at[pl.ds(cb + b * ck, ck)],
                                   cidx.at[b], sem_i.at[b]) for b in range(NB)]
            st = []
            for b in range(NB):
                ld[b].wait()
                st.append(pltpu.async_copy(ones_v, acc.at[cidx.at[b]],
                                           sem_s.at[b], add=True))
            for b in range(NB):
                st[b].wait()

        @pl.loop(ngrp * NB, ccnt)
        def _(k):
            pltpu.sync_copy(col_hbm.at[pl.ds(e + (cstart + k) * ck, ck)],
                            cidx.at[0])
            pltpu.sync_copy(ones_v, acc.at[cidx.at[0]], add=True)

        plsc.subcore_barrier()

        @pl.loop(0, nzc)
        def _(r):
            pltpu.sync_copy(acc.at[pl.ds(zb + r * zch, zch)],
                            out_hbm.at[pl.ds(cid * n + zb + r * zch, zch),
                                       pl.ds(0, 16)])

    return deg_kernel(eidx)


def _sc_agg(eidx, g, n, hf, nb=NB, ck=2 * CHUNK):
    e = eidx.shape[0] // 2
    nchunks = e // ck
    assert nchunks * ck == e and hf % LANES == 0
    span, tail, zch = _row_split(n)
    while zch * hf > 5120 and zch % 16 == 0:
        zch //= 2
    base, extra = _chunk_split(nchunks, NS)

    @functools.partial(
        pl.kernel,
        out_type=jax.ShapeDtypeStruct((2 * n, hf), _F32),
        mesh=plsc.VectorSubcoreMesh(core_axis_name="c", subcore_axis_name="s"),
        scratch_types=[
            pltpu.VMEM((nb, ck), jnp.int32),
            pltpu.VMEM((nb, ck), jnp.int32),
            pltpu.VMEM((nb, ck, hf), _F32),
            pltpu.VMEM((8, hf), _F32),
            pltpu.VMEM_SHARED((n, hf), _F32),
            pltpu.SemaphoreType.DMA((nb,)),
            pltpu.SemaphoreType.DMA((nb,)),
            pltpu.SemaphoreType.DMA((nb,)),
            pltpu.SemaphoreType.DMA((nb,)),
            pltpu.SemaphoreType.DMA,
        ],
        compiler_params=pltpu.CompilerParams(use_tc_tiling_on_sc=False),
    )
    def agg_kernel(eidx_hbm, g_hbm, s_hbm,
                   ridx, cidx, gbuf, zbuf, acc,
                   sem_ir, sem_ic, sem_g, sem_s, sem_z):
        cid = lax.axis_index("c")
        sid = lax.axis_index("s")
        gwin = g_hbm.at[pl.ds(cid * n, n)]
        _fill2d(zbuf, 8, hf, 0.0)
        zb = sid * span
        cnt = jnp.minimum(span, n - zb)
        nz8 = cnt // 8
        nzc = cnt // zch

        @pl.loop(0, nz8)
        def _(r):
            pltpu.async_copy(zbuf, acc.at[pl.ds(zb + r * 8, 8)], sem_z)

        @pl.loop(0, nz8)
        def _(r):
            pltpu.make_async_copy(zbuf, acc.at[pl.ds(zb + r * 8, 8)],
                                  sem_z).wait()

        plsc.subcore_barrier()
        cstart, ccnt = _worker_chunks(sid, base, extra)
        ngrp = ccnt // nb

        @pl.loop(0, ngrp)
        def _(grp):
            cb = (cstart + grp * nb) * ck
            ldr = [pltpu.async_copy(
                eidx_hbm.at[pl.ds(cb + b * ck, ck)],
                ridx.at[b], sem_ir.at[b]) for b in range(nb)]
            ldc = [pltpu.async_copy(
                eidx_hbm.at[pl.ds(e + cb + b * ck, ck)],
                cidx.at[b], sem_ic.at[b]) for b in range(nb)]
            gth = []
            for b in range(nb):
                ldr[b].wait()
                gth.append(pltpu.async_copy(gwin.at[ridx.at[b]], gbuf.at[b],
                                            sem_g.at[b]))
            sct = []
            for b in range(nb):
                gth[b].wait()
                ldc[b].wait()
                sct.append(pltpu.async_copy(gbuf.at[b], acc.at[cidx.at[b]],
                                            sem_s.at[b], add=True))
            for b in range(nb):
                sct[b].wait()

        @pl.loop(ngrp * nb, ccnt)
        def _(k):
            kb = (cstart + k) * ck
            pltpu.sync_copy(eidx_hbm.at[pl.ds(kb, ck)], ridx.at[0])
            pltpu.sync_copy(eidx_hbm.at[pl.ds(e + kb, ck)], cidx.at[0])
            pltpu.sync_copy(gwin.at[ridx.at[0]], gbuf.at[0])
            pltpu.sync_copy(gbuf.at[0], acc.at[cidx.at[0]], add=True)

        plsc.subcore_barrier()

        @pl.loop(0, nzc)
        def _(r):
            pltpu.sync_copy(acc.at[pl.ds(zb + r * zch, zch)],
                            s_hbm.at[pl.ds(cid * n + zb + r * zch, zch)])

    return agg_kernel(eidx, g)


def _sc_agg_es(eidx, g, n):
    e = eidx.shape[0] // 2
    ck = CHUNK
    nchunks = e // ck
    assert nchunks * ck == e
    span, tail, zch = _row_split(n)
    while zch * 128 > 5120 and zch % 16 == 0:
        zch //= 2
    base, extra = _chunk_split(nchunks, NC * NS)
    nb = 3

    @functools.partial(
        pl.kernel,
        out_type=jax.ShapeDtypeStruct((2 * n, 128), _F32),
        mesh=plsc.VectorSubcoreMesh(core_axis_name="c", subcore_axis_name="s"),
        scratch_types=[
            pltpu.VMEM((nb, ck), jnp.int32),
            pltpu.VMEM((nb, ck), jnp.int32),
            pltpu.VMEM((nb, ck, 128), _F32),
            pltpu.VMEM((8, 128), _F32),
            pltpu.VMEM_SHARED((n, 128), _F32),
            pltpu.SemaphoreType.DMA((nb,)),
            pltpu.SemaphoreType.DMA((nb,)),
            pltpu.SemaphoreType.DMA((nb,)),
            pltpu.SemaphoreType.DMA((nb,)),
            pltpu.SemaphoreType.DMA,
        ],
        compiler_params=pltpu.CompilerParams(use_tc_tiling_on_sc=False),
    )
    def agg_kernel(eidx_hbm, g_hbm, s_hbm,
                   ridx, cidx, gbuf, zbuf, acc,
                   sem_ir, sem_ic, sem_g, sem_s, sem_z):
        cid = lax.axis_index("c")
        sid = lax.axis_index("s")
        _fill2d(zbuf, 8, 128, 0.0)
        zb = sid * span
        cnt = jnp.minimum(span, n - zb)
        nz8 = cnt // 8
        nzc = cnt // zch

        @pl.loop(0, nz8)
        def _(r):
            pltpu.async_copy(zbuf, acc.at[pl.ds(zb + r * 8, 8)], sem_z)

        @pl.loop(0, nz8)
        def _(r):
            pltpu.make_async_copy(zbuf, acc.at[pl.ds(zb + r * 8, 8)],
                                  sem_z).wait()

        plsc.subcore_barrier()
        cstart, ccnt = _worker_chunks(cid * NS + sid, base, extra)
        ngrp = ccnt // nb

        @pl.loop(0, ngrp)
        def _(grp):
            cb = (cstart + grp * nb) * ck
            ldr = [pltpu.async_copy(eidx_hbm.at[pl.ds(cb + b * ck, ck)],
                                    ridx.at[b], sem_ir.at[b])
                   for b in range(nb)]
            ldc = [pltpu.async_copy(eidx_hbm.at[pl.ds(e + cb + b * ck, ck)],
                                    cidx.at[b], sem_ic.at[b])
                   for b in range(nb)]
            gth = []
            for b in range(nb):
                ldr[b].wait()
                gth.append(pltpu.async_copy(g_hbm.at[ridx.at[b]], gbuf.at[b],
                                            sem_g.at[b]))
            sct = []
            for b in range(nb):
                gth[b].wait()
                ldc[b].wait()
                sct.append(pltpu.async_copy(gbuf.at[b], acc.at[cidx.at[b]],
                                            sem_s.at[b], add=True))
            for b in range(nb):
                sct[b].wait()

        @pl.loop(ngrp * nb, ccnt)
        def _(k):
            kb = (cstart + k) * ck
            pltpu.sync_copy(eidx_hbm.at[pl.ds(kb, ck)], ridx.at[0])
            pltpu.sync_copy(eidx_hbm.at[pl.ds(e + kb, ck)], cidx.at[0])
            pltpu.sync_copy(g_hbm.at[ridx.at[0]], gbuf.at[0])
            pltpu.sync_copy(gbuf.at[0], acc.at[cidx.at[0]], add=True)

        plsc.subcore_barrier()

        @pl.loop(0, nzc)
        def _(r):
            pltpu.sync_copy(acc.at[pl.ds(zb + r * zch, zch)],
                            s_hbm.at[pl.ds(cid * n + zb + r * zch, zch)])

    return agg_kernel(eidx, g)


_BN = 1000


def _tc_prolog(x, degp, n):
    din = x.shape[1]
    nb = n // _BN

    def body(x_ref, d0, d1, g_ref, dinv_ref):
        dinv = lax.rsqrt(d0[:, 0:1] + d1[:, 0:1] + 1.0)
        g_ref[...] = x_ref[...] * dinv
        dinv_ref[...] = dinv

    return pl.pallas_call(
        body,
        grid=(nb,),
        in_specs=[
            pl.BlockSpec((_BN, din), lambda i: (i, 0)),
            pl.BlockSpec((_BN, 128), lambda i: (i, 0)),
            pl.BlockSpec((_BN, 128), lambda i: (i + nb, 0)),
        ],
        out_specs=[
            pl.BlockSpec((_BN, din), lambda i: (i, 0)),
            pl.BlockSpec((_BN, 1), lambda i: (i, 0)),
        ],
        out_shape=[
            jax.ShapeDtypeStruct((n, din), _F32),
            jax.ShapeDtypeStruct((n, 1), _F32),
        ],
    )(x, degp, degp)


def _tc_layer1(s, g, dinv, w, b, n):
    din = g.shape[1]
    dout = w.shape[1]
    ho = dout // 2
    nb = n // _BN

    def body(p0, p1, g0, dv, w_ref, b_ref, out):
        t = (p0[...] + p1[...] + g0[...]) * dv[...]
        hh = jnp.dot(t, w_ref[...], precision=_HI,
                     preferred_element_type=_F32) + b_ref[...]
        hh = jnp.maximum(hh, 0.0)
        gg = hh * dv[...]
        out[0] = gg[:, :ho]
        out[1] = gg[:, ho:]

    return pl.pallas_call(
        body,
        grid=(nb,),
        in_specs=[
            pl.BlockSpec((_BN, din), lambda i: (i, 0)),
            pl.BlockSpec((_BN, din), lambda i: (i + nb, 0)),
            pl.BlockSpec((_BN, din), lambda i: (i, 0)),
            pl.BlockSpec((_BN, 1), lambda i: (i, 0)),
            pl.BlockSpec(w.shape, lambda i: (0, 0)),
            pl.BlockSpec(b.shape, lambda i: (0, 0)),
        ],
        out_specs=pl.BlockSpec((2, _BN, ho), lambda i: (0, i, 0)),
        out_shape=jax.ShapeDtypeStruct((2, n, ho), _F32),
    )(s, s, g, dinv, w, b)


def _tc_layer(s, g, dinv, w, b, n, w2=None):
    hf = s.shape[1]
    dout = (w2 if w2 is not None else w).shape[1]
    ho = dout // 2
    nb = n // _BN

    def body(s0, s1, g0, g1, dv, w_ref, b_ref, *rest):
        if w2 is not None:
            w2_ref, out = rest
        else:
            (out,) = rest
        t = jnp.concatenate([s0[...] + g0[...], s1[...] + g1[...]], axis=1)
        t = t * dv[...]
        hh = jnp.dot(t, w_ref[...], precision=_HI,
                     preferred_element_type=_F32) + b_ref[...]
        hh = jnp.maximum(hh, 0.0)
        if w2 is not None:
            hh = jnp.dot(hh, w2_ref[...], precision=_HI,
                         preferred_element_type=_F32)
        gg = hh * dv[...]
        out[0] = gg[:, :ho]
        out[1] = gg[:, ho:]

    in_specs = [
        pl.BlockSpec((_BN, hf), lambda i: (i, 0)),
        pl.BlockSpec((_BN, hf), lambda i: (i + nb, 0)),
        pl.BlockSpec((_BN, hf), lambda i: (i, 0)),
        pl.BlockSpec((_BN, hf), lambda i: (i + nb, 0)),
        pl.BlockSpec((_BN, 1), lambda i: (i, 0)),
        pl.BlockSpec(w.shape, lambda i: (0, 0)),
        pl.BlockSpec(b.shape, lambda i: (0, 0)),
    ]
    args = [s, s, g, g, dinv, w, b]
    if w2 is not None:
        in_specs.append(pl.BlockSpec(w2.shape, lambda i: (0, 0)))
        args.append(w2)
    return pl.pallas_call(
        body,
        grid=(nb,),
        in_specs=in_specs,
        out_specs=pl.BlockSpec((2, _BN, ho), lambda i: (0, i, 0)),
        out_shape=jax.ShapeDtypeStruct((2, n, ho), _F32),
    )(*args)


def _tc_epilog(s, g, dinv, b34, n):
    hf = s.shape[1]
    nb = n // _BN

    def body(s0, s1, g0, g1, dv, b_ref, out):
        t = jnp.concatenate([s0[...] + g0[...], s1[...] + g1[...]], axis=1)
        out[...] = t * dv[...] + b_ref[...]

    return pl.pallas_call(
        body,
        grid=(nb,),
        in_specs=[
            pl.BlockSpec((_BN, hf), lambda i: (i, 0)),
            pl.BlockSpec((_BN, hf), lambda i: (i + nb, 0)),
            pl.BlockSpec((_BN, hf), lambda i: (i, 0)),
            pl.BlockSpec((_BN, hf), lambda i: (i + nb, 0)),
            pl.BlockSpec((_BN, 1), lambda i: (i, 0)),
            pl.BlockSpec(b34.shape, lambda i: (0, 0)),
        ],
        out_specs=pl.BlockSpec((_BN, 2 * hf), lambda i: (i, 0)),
        out_shape=jax.ShapeDtypeStruct((n, 2 * hf), _F32),
    )(s, s, g, g, dinv, b34)


def _tc_fold(w3, wout, b3, bout):
    def body(w3_ref, wo_ref, b3_ref, bo_ref, w34_ref, b34_ref):
        w34_ref[...] = jnp.dot(w3_ref[...], wo_ref[...], precision=_HI,
                               preferred_element_type=_F32)
        b34_ref[...] = jnp.dot(b3_ref[...], wo_ref[...], precision=_HI,
                               preferred_element_type=_F32) + bo_ref[...]

    return pl.pallas_call(
        body,
        out_shape=[
            jax.ShapeDtypeStruct((w3.shape[0], wout.shape[1]), _F32),
            jax.ShapeDtypeStruct((1, wout.shape[1]), _F32),
        ],
    )(w3, wout, b3, bout)


def kernel(x, edge_index, W1, b1, W2, b2, W3, b3, Wout, bout):
    n, din = x.shape
    e = edge_index.shape[1]
    assert e % (2 * CHUNK) == 0
    eidx = edge_index.reshape(2 * e)

    b1r = b1.reshape(1, -1)
    b2r = b2.reshape(1, -1)
    b3r = b3.reshape(1, -1)
    boutr = bout.reshape(1, -1)

    w34, b34 = _tc_fold(W3, Wout, b3r, boutr)

    degp = _sc_deg(eidx, n)
    g1, dinv = _tc_prolog(x, degp, n)
    s1 = _sc_agg_es(eidx, g1, n)

    g2_3d = _tc_layer1(s1, g1, dinv, W1, b1r, n)
    g2 = g2_3d.reshape(2 * n, W1.shape[1] // 2)
    s2 = _sc_agg(eidx, g2, n, W1.shape[1] // 2, nb=3, ck=CHUNK)

    g3_3d = _tc_layer(s2, g2, dinv, W2, b2r, n, w2=w34)
    g3 = g3_3d.reshape(2 * n, Wout.shape[1] // 2)
    s3 = _sc_agg(eidx, g3, n, Wout.shape[1] // 2)

    return _tc_epilog(s3, g3, dinv, b34, n)


# speedup vs baseline: 23.3090x; 1.0084x over previous
import functools
import math

import jax
import jax.numpy as jnp
from jax import lax
from jax.experimental import pallas as pl
from jax.experimental.pallas import tpu as pltpu
from jax.experimental.pallas import tpu_sc as plsc

NC = 2
NS = 16
LANES = 16
CHUNK = 128

_HI = lax.Precision.HIGHEST
_F32 = jnp.float32


def _row_split(n):
    for mult in (80, 40, 16, 8):
        span = -(-(-(-n // NS)) // mult) * mult
        tail = n - (NS - 1) * span
        if 0 < tail <= span and tail % 8 == 0:
            return span, tail, math.gcd(span, tail)
    raise ValueError(f"no 8-aligned row split for n={n}")


def _fill2d(ref, rows, width, val):
    @pl.loop(0, rows)
    def _(i):
        @pl.loop(0, width, step=LANES)
        def _(j):
            ref[i, pl.ds(j, LANES)] = jnp.full((LANES,), val, _F32)


NB = 4


def _chunk_split(nchunks, nworkers):
    base, extra = divmod(nchunks, nworkers)
    return base, extra


def _worker_chunks(wid, base, extra):
    start = wid * base + jnp.minimum(wid, extra)
    cnt = base + jnp.where(wid < extra, 1, 0)
    return start, cnt


def _sc_deg(eidx, n):
    e = eidx.shape[0] // 2
    ck = 2 * CHUNK
    nchunks = e // ck
    assert nchunks * ck == e
    span, tail, zch = _row_split(n)
    base, extra = _chunk_split(nchunks, NC * NS)

    @functools.partial(
        pl.kernel,
        out_type=jax.ShapeDtypeStruct((2 * n, 128), _F32),
        mesh=plsc.VectorSubcoreMesh(core_axis_name="c", subcore_axis_name="s"),
        scratch_types=[
            pltpu.VMEM((ck, 16), _F32),
            pltpu.VMEM((NB, ck), jnp.int32),
            pltpu.VMEM((zch, 16), _F32),
            pltpu.VMEM_SHARED((n, 16), _F32),
            pltpu.SemaphoreType.DMA((NB,)),
            pltpu.SemaphoreType.DMA((NB,)),
        ],
        compiler_params=pltpu.CompilerParams(use_tc_tiling_on_sc=False),
    )
    def deg_kernel(col_hbm, out_hbm, ones_v, cidx, zbuf, acc, sem_i, sem_s):
        cid = lax.axis_index("c")
        sid = lax.axis_index("s")
        _fill2d(ones_v, ck, 16, 1.0)
        _fill2d(zbuf, zch, 16, 0.0)
        zb = sid * span
        nzc = jnp.minimum(span, n - zb) // zch

        @pl.loop(0, nzc)
        def _(r):
            pltpu.sync_copy(zbuf, acc.at[pl.ds(zb + r * zch, zch)])

        plsc.subcore_barrier()
        cstart, ccnt = _worker_chunks(cid * NS + sid, base, extra)
        ngrp = ccnt // NB

        @pl.loop(0, ngrp)
        def _(g):
            cb = e + (cstart + g * NB) * ck
            ld = [pltpu.async_copy(col_hbm.at[pl.ds(cb + b * ck, ck)],
                                   cidx.at[b], sem_i.at[b]) for b in range(NB)]
            st = []
            for b in range(NB):
                ld[b].wait()
                st.append(pltpu.async_copy(ones_v, acc.at[cidx.at[b]],
                                           sem_s.at[b], add=True))
            for b in range(NB):
                st[b].wait()

        @pl.loop(ngrp * NB, ccnt)
        def _(k):
            pltpu.sync_copy(col_hbm.at[pl.ds(e + (cstart + k) * ck, ck)],
                            cidx.at[0])
            pltpu.sync_copy(ones_v, acc.at[cidx.at[0]], add=True)

        plsc.subcore_barrier()

        @pl.loop(0, nzc)
        def _(r):
            pltpu.sync_copy(acc.at[pl.ds(zb + r * zch, zch)],
                            out_hbm.at[pl.ds(cid * n + zb + r * zch, zch),
                                       pl.ds(0, 16)])

    return deg_kernel(eidx)


def _sc_agg(eidx, g, n, hf, nb=NB, ck=2 * CHUNK):
    e = eidx.shape[0] // 2
    nchunks = e // ck
    assert nchunks * ck == e and hf % LANES == 0
    span, tail, zch = _row_split(n)
    while zch * hf > 5120 and zch % 16 == 0:
        zch //= 2
    base, extra = _chunk_split(nchunks, NS)

    @functools.partial(
        pl.kernel,
        out_type=jax.ShapeDtypeStruct((2 * n, hf), _F32),
        mesh=plsc.VectorSubcoreMesh(core_axis_name="c", subcore_axis_name="s"),
        scratch_types=[
            pltpu.VMEM((nb, ck), jnp.int32),
            pltpu.VMEM((nb, ck), jnp.int32),
            pltpu.VMEM((nb, ck, hf), _F32),
            pltpu.VMEM((8, hf), _F32),
            pltpu.VMEM_SHARED((n, hf), _F32),
            pltpu.SemaphoreType.DMA((nb,)),
            pltpu.SemaphoreType.DMA((nb,)),
            pltpu.SemaphoreType.DMA((nb,)),
            pltpu.SemaphoreType.DMA((nb,)),
            pltpu.SemaphoreType.DMA,
        ],
        compiler_params=pltpu.CompilerParams(use_tc_tiling_on_sc=False),
    )
    def agg_kernel(eidx_hbm, g_hbm, s_hbm,
                   ridx, cidx, gbuf, zbuf, acc,
                   sem_ir, sem_ic, sem_g, sem_s, sem_z):
        cid = lax.axis_index("c")
        sid = lax.axis_index("s")
        gwin = g_hbm.at[pl.ds(cid * n, n)]
        _fill2d(zbuf, 8, hf, 0.0)
        zb = sid * span
        cnt = jnp.minimum(span, n - zb)
        nz8 = cnt // 8
        nzc = cnt // zch

        @pl.loop(0, nz8)
        def _(r):
            pltpu.async_copy(zbuf, acc.at[pl.ds(zb + r * 8, 8)], sem_z)

        @pl.loop(0, nz8)
        def _(r):
            pltpu.make_async_copy(zbuf, acc.at[pl.ds(zb + r * 8, 8)],
                                  sem_z).wait()

        plsc.subcore_barrier()
        cstart, ccnt = _worker_chunks(sid, base, extra)
        ngrp = ccnt // nb

        @pl.loop(0, ngrp)
        def _(grp):
            cb = (cstart + grp * nb) * ck
            ldr = [pltpu.async_copy(
                eidx_hbm.at[pl.ds(cb + b * ck, ck)],
                ridx.at[b], sem_ir.at[b]) for b in range(nb)]
            ldc = [pltpu.async_copy(
                eidx_hbm.at[pl.ds(e + cb + b * ck, ck)],
                cidx.at[b], sem_ic.at[b]) for b in range(nb)]
            gth = []
            for b in range(nb):
                ldr[b].wait()
                gth.append(pltpu.async_copy(gwin.at[ridx.at[b]], gbuf.at[b],
                                            sem_g.at[b]))
            sct = []
            for b in range(nb):
                gth[b].wait()
                ldc[b].wait()
                sct.append(pltpu.async_copy(gbuf.at[b], acc.at[cidx.at[b]],
                                            sem_s.at[b], add=True))
            for b in range(nb):
                sct[b].wait()

        @pl.loop(ngrp * nb, ccnt)
        def _(k):
            kb = (cstart + k) * ck
            pltpu.sync_copy(eidx_hbm.at[pl.ds(kb, ck)], ridx.at[0])
            pltpu.sync_copy(eidx_hbm.at[pl.ds(e + kb, ck)], cidx.at[0])
            pltpu.sync_copy(gwin.at[ridx.at[0]], gbuf.at[0])
            pltpu.sync_copy(gbuf.at[0], acc.at[cidx.at[0]], add=True)

        plsc.subcore_barrier()

        @pl.loop(0, nzc)
        def _(r):
            pltpu.sync_copy(acc.at[pl.ds(zb + r * zch, zch)],
                            s_hbm.at[pl.ds(cid * n + zb + r * zch, zch)])

    return agg_kernel(eidx, g)


def _sc_agg_es(eidx, g, n):
    e = eidx.shape[0] // 2
    ck = CHUNK
    nchunks = e // ck
    assert nchunks * ck == e
    span, tail, zch = _row_split(n)
    while zch * 128 > 5120 and zch % 16 == 0:
        zch //= 2
    base, extra = _chunk_split(nchunks, NC * NS)
    nb = 3

    @functools.partial(
        pl.kernel,
        out_type=jax.ShapeDtypeStruct((2 * n, 128), _F32),
        mesh=plsc.VectorSubcoreMesh(core_axis_name="c", subcore_axis_name="s"),
        scratch_types=[
            pltpu.VMEM((nb, ck), jnp.int32),
            pltpu.VMEM((nb, ck), jnp.int32),
            pltpu.VMEM((nb, ck, 128), _F32),
            pltpu.VMEM((8, 128), _F32),
            pltpu.VMEM_SHARED((n, 128), _F32),
            pltpu.SemaphoreType.DMA((nb,)),
            pltpu.SemaphoreType.DMA((nb,)),
            pltpu.SemaphoreType.DMA((nb,)),
            pltpu.SemaphoreType.DMA((nb,)),
            pltpu.SemaphoreType.DMA,
        ],
        compiler_params=pltpu.CompilerParams(use_tc_tiling_on_sc=False),
    )
    def agg_kernel(eidx_hbm, g_hbm, s_hbm,
                   ridx, cidx, gbuf, zbuf, acc,
                   sem_ir, sem_ic, sem_g, sem_s, sem_z):
        cid = lax.axis_index("c")
        sid = lax.axis_index("s")
        _fill2d(zbuf, 8, 128, 0.0)
        zb = sid * span
        cnt = jnp.minimum(span, n - zb)
        nz8 = cnt // 8
        nzc = cnt // zch

        @pl.loop(0, nz8)
        def _(r):
            pltpu.async_copy(zbuf, acc.at[pl.ds(zb + r * 8, 8)], sem_z)

        @pl.loop(0, nz8)
        def _(r):
            pltpu.make_async_copy(zbuf, acc.at[pl.ds(zb + r * 8, 8)],
                                  sem_z).wait()

        plsc.subcore_barrier()
        cstart, ccnt = _worker_chunks(cid * NS + sid, base, extra)
        ngrp = ccnt // nb

        @pl.loop(0, ngrp)
        def _(grp):
            cb = (cstart + grp * nb) * ck
            ldr = [pltpu.async_copy(eidx_hbm.at[pl.ds(cb + b * ck, ck)],
                                    ridx.at[b], sem_ir.at[b])
                   for b in range(nb)]
            ldc = [pltpu.async_copy(eidx_hbm.at[pl.ds(e + cb + b * ck, ck)],
                                    cidx.at[b], sem_ic.at[b])
                   for b in range(nb)]
            gth = []
            for b in range(nb):
                ldr[b].wait()
                gth.append(pltpu.async_copy(g_hbm.at[ridx.at[b]], gbuf.at[b],
                                            sem_g.at[b]))
            sct = []
            for b in range(nb):
                gth[b].wait()
                ldc[b].wait()
                sct.append(pltpu.async_copy(gbuf.at[b], acc.at[cidx.at[b]],
                                            sem_s.at[b], add=True))
            for b in range(nb):
                sct[b].wait()

        @pl.loop(ngrp * nb, ccnt)
        def _(k):
            kb = (cstart + k) * ck
            pltpu.sync_copy(eidx_hbm.at[pl.ds(kb, ck)], ridx.at[0])
            pltpu.sync_copy(eidx_hbm.at[pl.ds(e + kb, ck)], cidx.at[0])
            pltpu.sync_copy(g_hbm.at[ridx.at[0]], gbuf.at[0])
            pltpu.sync_copy(gbuf.at[0], acc.at[cidx.at[0]], add=True)

        plsc.subcore_barrier()

        @pl.loop(0, nzc)
        def _(r):
            pltpu.sync_copy(acc.at[pl.ds(zb + r * zch, zch)],
                            s_hbm.at[pl.ds(cid * n + zb + r * zch, zch)])

    return agg_kernel(eidx, g)


_BN = 2000


def _tc_prolog(x, degp, n):
    din = x.shape[1]
    nb = n // _BN

    def body(x_ref, d0, d1, g_ref, dinv_ref):
        dinv = lax.rsqrt(d0[:, 0:1] + d1[:, 0:1] + 1.0)
        g_ref[...] = x_ref[...] * dinv
        dinv_ref[...] = dinv

    return pl.pallas_call(
        body,
        grid=(nb,),
        in_specs=[
            pl.BlockSpec((_BN, din), lambda i: (i, 0)),
            pl.BlockSpec((_BN, 128), lambda i: (i, 0)),
            pl.BlockSpec((_BN, 128), lambda i: (i + nb, 0)),
        ],
        out_specs=[
            pl.BlockSpec((_BN, din), lambda i: (i, 0)),
            pl.BlockSpec((_BN, 1), lambda i: (i, 0)),
        ],
        out_shape=[
            jax.ShapeDtypeStruct((n, din), _F32),
            jax.ShapeDtypeStruct((n, 1), _F32),
        ],
    )(x, degp, degp)


def _tc_layer1(s, g, dinv, w, b, n):
    din = g.shape[1]
    dout = w.shape[1]
    ho = dout // 2
    nb = n // _BN

    def body(p0, p1, g0, dv, w_ref, b_ref, out):
        t = (p0[...] + p1[...] + g0[...]) * dv[...]
        hh = jnp.dot(t, w_ref[...], precision=_HI,
                     preferred_element_type=_F32) + b_ref[...]
        hh = jnp.maximum(hh, 0.0)
        gg = hh * dv[...]
        out[0] = gg[:, :ho]
        out[1] = gg[:, ho:]

    return pl.pallas_call(
        body,
        grid=(nb,),
        in_specs=[
            pl.BlockSpec((_BN, din), lambda i: (i, 0)),
            pl.BlockSpec((_BN, din), lambda i: (i + nb, 0)),
            pl.BlockSpec((_BN, din), lambda i: (i, 0)),
            pl.BlockSpec((_BN, 1), lambda i: (i, 0)),
            pl.BlockSpec(w.shape, lambda i: (0, 0)),
            pl.BlockSpec(b.shape, lambda i: (0, 0)),
        ],
        out_specs=pl.BlockSpec((2, _BN, ho), lambda i: (0, i, 0)),
        out_shape=jax.ShapeDtypeStruct((2, n, ho), _F32),
    )(s, s, g, dinv, w, b)


def _tc_layer(s, g, dinv, w, b, n, w2=None):
    hf = s.shape[1]
    dout = (w2 if w2 is not None else w).shape[1]
    ho = dout // 2
    nb = n // _BN

    def body(s0, s1, g0, g1, dv, w_ref, b_ref, *rest):
        if w2 is not None:
            w2_ref, out = rest
        else:
            (out,) = rest
        t = jnp.concatenate([s0[...] + g0[...], s1[...] + g1[...]], axis=1)
        t = t * dv[...]
        hh = jnp.dot(t, w_ref[...], precision=_HI,
                     preferred_element_type=_F32) + b_ref[...]
        hh = jnp.maximum(hh, 0.0)
        if w2 is not None:
            hh = jnp.dot(hh, w2_ref[...], precision=_HI,
                         preferred_element_type=_F32)
        gg = hh * dv[...]
        out[0] = gg[:, :ho]
        out[1] = gg[:, ho:]

    in_specs = [
        pl.BlockSpec((_BN, hf), lambda i: (i, 0)),
        pl.BlockSpec((_BN, hf), lambda i: (i + nb, 0)),
        pl.BlockSpec((_BN, hf), lambda i: (i, 0)),
        pl.BlockSpec((_BN, hf), lambda i: (i + nb, 0)),
        pl.BlockSpec((_BN, 1), lambda i: (i, 0)),
        pl.BlockSpec(w.shape, lambda i: (0, 0)),
        pl.BlockSpec(b.shape, lambda i: (0, 0)),
    ]
    args = [s, s, g, g, dinv, w, b]
    if w2 is not None:
        in_specs.append(pl.BlockSpec(w2.shape, lambda i: (0, 0)))
        args.append(w2)
    return pl.pallas_call(
        body,
        grid=(nb,),
        in_specs=in_specs,
        out_specs=pl.BlockSpec((2, _BN, ho), lambda i: (0, i, 0)),
        out_shape=jax.ShapeDtypeStruct((2, n, ho), _F32),
    )(*args)


def _tc_epilog(s, g, dinv, b34, n):
    hf = s.shape[1]
    nb = n // _BN

    def body(s0, s1, g0, g1, dv, b_ref, out):
        t = jnp.concatenate([s0[...] + g0[...], s1[...] + g1[...]], axis=1)
        out[...] = t * dv[...] + b_ref[...]

    return pl.pallas_call(
        body,
        grid=(nb,),
        in_specs=[
            pl.BlockSpec((_BN, hf), lambda i: (i, 0)),
            pl.BlockSpec((_BN, hf), lambda i: (i + nb, 0)),
            pl.BlockSpec((_BN, hf), lambda i: (i, 0)),
            pl.BlockSpec((_BN, hf), lambda i: (i + nb, 0)),
            pl.BlockSpec((_BN, 1), lambda i: (i, 0)),
            pl.BlockSpec(b34.shape, lambda i: (0, 0)),
        ],
        out_specs=pl.BlockSpec((_BN, 2 * hf), lambda i: (i, 0)),
        out_shape=jax.ShapeDtypeStruct((n, 2 * hf), _F32),
    )(s, s, g, g, dinv, b34)


def _tc_fold(w3, wout, b3, bout):
    def body(w3_ref, wo_ref, b3_ref, bo_ref, w34_ref, b34_ref):
        w34_ref[...] = jnp.dot(w3_ref[...], wo_ref[...], precision=_HI,
                               preferred_element_type=_F32)
        b34_ref[...] = jnp.dot(b3_ref[...], wo_ref[...], precision=_HI,
                               preferred_element_type=_F32) + bo_ref[...]

    return pl.pallas_call(
        body,
        out_shape=[
            jax.ShapeDtypeStruct((w3.shape[0], wout.shape[1]), _F32),
            jax.ShapeDtypeStruct((1, wout.shape[1]), _F32),
        ],
    )(w3, wout, b3, bout)


def kernel(x, edge_index, W1, b1, W2, b2, W3, b3, Wout, bout):
    n, din = x.shape
    e = edge_index.shape[1]
    assert e % (2 * CHUNK) == 0
    eidx = edge_index.reshape(2 * e)

    b1r = b1.reshape(1, -1)
    b2r = b2.reshape(1, -1)
    b3r = b3.reshape(1, -1)
    boutr = bout.reshape(1, -1)

    w34, b34 = _tc_fold(W3, Wout, b3r, boutr)

    degp = _sc_deg(eidx, n)
    g1, dinv = _tc_prolog(x, degp, n)
    s1 = _sc_agg_es(eidx, g1, n)

    g2_3d = _tc_layer1(s1, g1, dinv, W1, b1r, n)
    g2 = g2_3d.reshape(2 * n, W1.shape[1] // 2)
    s2 = _sc_agg(eidx, g2, n, W1.shape[1] // 2, nb=3, ck=CHUNK)

    g3_3d = _tc_layer(s2, g2, dinv, W2, b2r, n, w2=w34)
    g3 = g3_3d.reshape(2 * n, Wout.shape[1] // 2)
    s3 = _sc_agg(eidx, g3, n, Wout.shape[1] // 2)

    return _tc_epilog(s3, g3, dinv, b34, n)


# speedup vs baseline: 23.5367x; 1.0098x over previous
import functools
import math

import jax
import jax.numpy as jnp
from jax import lax
from jax.experimental import pallas as pl
from jax.experimental.pallas import tpu as pltpu
from jax.experimental.pallas import tpu_sc as plsc

NC = 2
NS = 16
LANES = 16
CHUNK = 128

_HI = lax.Precision.HIGHEST
_F32 = jnp.float32


def _row_split(n):
    for mult in (80, 40, 16, 8):
        span = -(-(-(-n // NS)) // mult) * mult
        tail = n - (NS - 1) * span
        if 0 < tail <= span and tail % 8 == 0:
            return span, tail, math.gcd(span, tail)
    raise ValueError(f"no 8-aligned row split for n={n}")


def _fill2d(ref, rows, width, val):
    @pl.loop(0, rows)
    def _(i):
        @pl.loop(0, width, step=LANES)
        def _(j):
            ref[i, pl.ds(j, LANES)] = jnp.full((LANES,), val, _F32)


NB = 4


def _chunk_split(nchunks, nworkers):
    base, extra = divmod(nchunks, nworkers)
    return base, extra


def _worker_chunks(wid, base, extra):
    start = wid * base + jnp.minimum(wid, extra)
    cnt = base + jnp.where(wid < extra, 1, 0)
    return start, cnt


def _sc_deg(eidx, n):
    e = eidx.shape[0] // 2
    ck = 2 * CHUNK
    nchunks = e // ck
    assert nchunks * ck == e
    span, tail, zch = _row_split(n)
    base, extra = _chunk_split(nchunks, NC * NS)

    @functools.partial(
        pl.kernel,
        out_type=jax.ShapeDtypeStruct((2 * n, 128), _F32),
        mesh=plsc.VectorSubcoreMesh(core_axis_name="c", subcore_axis_name="s"),
        scratch_types=[
            pltpu.VMEM((ck, 16), _F32),
            pltpu.VMEM((NB, ck), jnp.int32),
            pltpu.VMEM((zch, 16), _F32),
            pltpu.VMEM_SHARED((n, 16), _F32),
            pltpu.SemaphoreType.DMA((NB,)),
            pltpu.SemaphoreType.DMA((NB,)),
        ],
        compiler_params=pltpu.CompilerParams(use_tc_tiling_on_sc=False),
    )
    def deg_kernel(col_hbm, out_hbm, ones_v, cidx, zbuf, acc, sem_i, sem_s):
        cid = lax.axis_index("c")
        sid = lax.axis_index("s")
        _fill2d(ones_v, ck, 16, 1.0)
        _fill2d(zbuf, zch, 16, 0.0)
        zb = sid * span
        nzc = jnp.minimum(span, n - zb) // zch

        @pl.loop(0, nzc)
        def _(r):
            pltpu.sync_copy(zbuf, acc.at[pl.ds(zb + r * zch, zch)])

        plsc.subcore_barrier()
        cstart, ccnt = _worker_chunks(cid * NS + sid, base, extra)
        ngrp = ccnt // NB

        @pl.loop(0, ngrp)
        def _(g):
            cb = e + (cstart + g * NB) * ck
            ld = [pltpu.async_copy(col_hbm.at[pl.ds(cb + b * ck, ck)],
                                   cidx.at[b], sem_i.at[b]) for b in range(NB)]
            st = []
            for b in range(NB):
                ld[b].wait()
                st.append(pltpu.async_copy(ones_v, acc.at[cidx.at[b]],
                                           sem_s.at[b], add=True))
            for b in range(NB):
                st[b].wait()

        @pl.loop(ngrp * NB, ccnt)
        def _(k):
            pltpu.sync_copy(col_hbm.at[pl.ds(e + (cstart + k) * ck, ck)],
                            cidx.at[0])
            pltpu.sync_copy(ones_v, acc.at[cidx.at[0]], add=True)

        plsc.subcore_barrier()

        @pl.loop(0, nzc)
        def _(r):
            pltpu.sync_copy(acc.at[pl.ds(zb + r * zch, zch)],
                            out_hbm.at[pl.ds(cid * n + zb + r * zch, zch),
                                       pl.ds(0, 16)])

    return deg_kernel(eidx)


def _sc_agg(eidx, g, n, hf, nb=NB, ck=2 * CHUNK):
    e = eidx.shape[0] // 2
    nchunks = e // ck
    assert nchunks * ck == e and hf % LANES == 0
    span, tail, zch = _row_split(n)
    while zch * hf > 5120 and zch % 16 == 0:
        zch //= 2
    base, extra = _chunk_split(nchunks, NS)

    @functools.partial(
        pl.kernel,
        out_type=jax.ShapeDtypeStruct((2 * n, hf), _F32),
        mesh=plsc.VectorSubcoreMesh(core_axis_name="c", subcore_axis_name="s"),
        scratch_types=[
            pltpu.VMEM((nb, ck), jnp.int32),
            pltpu.VMEM((nb, ck), jnp.int32),
            pltpu.VMEM((nb, ck, hf), _F32),
            pltpu.VMEM((8, hf), _F32),
            pltpu.VMEM_SHARED((n, hf), _F32),
            pltpu.SemaphoreType.DMA((nb,)),
            pltpu.SemaphoreType.DMA((nb,)),
            pltpu.SemaphoreType.DMA((nb,)),
            pltpu.SemaphoreType.DMA((nb,)),
            pltpu.SemaphoreType.DMA,
        ],
        compiler_params=pltpu.CompilerParams(use_tc_tiling_on_sc=False),
    )
    def agg_kernel(eidx_hbm, g_hbm, s_hbm,
                   ridx, cidx, gbuf, zbuf, acc,
                   sem_ir, sem_ic, sem_g, sem_s, sem_z):
        cid = lax.axis_index("c")
        sid = lax.axis_index("s")
        gwin = g_hbm.at[pl.ds(cid * n, n)]
        _fill2d(zbuf, 8, hf, 0.0)
        zb = sid * span
        cnt = jnp.minimum(span, n - zb)
        nz8 = cnt // 8
        nzc = cnt // zch

        @pl.loop(0, nz8)
        def _(r):
            pltpu.async_copy(zbuf, acc.at[pl.ds(zb + r * 8, 8)], sem_z)

        @pl.loop(0, nz8)
        def _(r):
            pltpu.make_async_copy(zbuf, acc.at[pl.ds(zb + r * 8, 8)],
                                  sem_z).wait()

        plsc.subcore_barrier()
        cstart, ccnt = _worker_chunks(sid, base, extra)
        ngrp = ccnt // nb

        @pl.loop(0, ngrp)
        def _(grp):
            cb = (cstart + grp * nb) * ck
            ldr = [pltpu.async_copy(
                eidx_hbm.at[pl.ds(cb + b * ck, ck)],
                ridx.at[b], sem_ir.at[b]) for b in range(nb)]
            ldc = [pltpu.async_copy(
                eidx_hbm.at[pl.ds(e + cb + b * ck, ck)],
                cidx.at[b], sem_ic.at[b]) for b in range(nb)]
            gth = []
            for b in range(nb):
                ldr[b].wait()
                gth.append(pltpu.async_copy(gwin.at[ridx.at[b]], gbuf.at[b],
                                            sem_g.at[b]))
            sct = []
            for b in range(nb):
                gth[b].wait()
                ldc[b].wait()
                sct.append(pltpu.async_copy(gbuf.at[b], acc.at[cidx.at[b]],
                                            sem_s.at[b], add=True))
            for b in range(nb):
                sct[b].wait()

        @pl.loop(ngrp * nb, ccnt)
        def _(k):
            kb = (cstart + k) * ck
            pltpu.sync_copy(eidx_hbm.at[pl.ds(kb, ck)], ridx.at[0])
            pltpu.sync_copy(eidx_hbm.at[pl.ds(e + kb, ck)], cidx.at[0])
            pltpu.sync_copy(gwin.at[ridx.at[0]], gbuf.at[0])
            pltpu.sync_copy(gbuf.at[0], acc.at[cidx.at[0]], add=True)

        plsc.subcore_barrier()

        @pl.loop(0, nzc)
        def _(r):
            pltpu.sync_copy(acc.at[pl.ds(zb + r * zch, zch)],
                            s_hbm.at[pl.ds(cid * n + zb + r * zch, zch)])

    return agg_kernel(eidx, g)


def _sc_agg_es(eidx, g, n):
    e = eidx.shape[0] // 2
    ck = CHUNK
    nchunks = e // ck
    assert nchunks * ck == e
    span, tail, zch = _row_split(n)
    while zch * 128 > 5120 and zch % 16 == 0:
        zch //= 2
    base, extra = _chunk_split(nchunks, NC * NS)
    nb = 3

    @functools.partial(
        pl.kernel,
        out_type=jax.ShapeDtypeStruct((2 * n, 128), _F32),
        mesh=plsc.VectorSubcoreMesh(core_axis_name="c", subcore_axis_name="s"),
        scratch_types=[
            pltpu.VMEM((nb, ck), jnp.int32),
            pltpu.VMEM((nb, ck), jnp.int32),
            pltpu.VMEM((nb, ck, 128), _F32),
            pltpu.VMEM((8, 128), _F32),
            pltpu.VMEM_SHARED((n, 128), _F32),
            pltpu.SemaphoreType.DMA((nb,)),
            pltpu.SemaphoreType.DMA((nb,)),
            pltpu.SemaphoreType.DMA((nb,)),
            pltpu.SemaphoreType.DMA((nb,)),
            pltpu.SemaphoreType.DMA,
        ],
        compiler_params=pltpu.CompilerParams(use_tc_tiling_on_sc=False),
    )
    def agg_kernel(eidx_hbm, g_hbm, s_hbm,
                   ridx, cidx, gbuf, zbuf, acc,
                   sem_ir, sem_ic, sem_g, sem_s, sem_z):
        cid = lax.axis_index("c")
        sid = lax.axis_index("s")
        _fill2d(zbuf, 8, 128, 0.0)
        zb = sid * span
        cnt = jnp.minimum(span, n - zb)
        nz8 = cnt // 8
        nzc = cnt // zch

        @pl.loop(0, nz8)
        def _(r):
            pltpu.async_copy(zbuf, acc.at[pl.ds(zb + r * 8, 8)], sem_z)

        @pl.loop(0, nz8)
        def _(r):
            pltpu.make_async_copy(zbuf, acc.at[pl.ds(zb + r * 8, 8)],
                                  sem_z).wait()

        plsc.subcore_barrier()
        cstart, ccnt = _worker_chunks(cid * NS + sid, base, extra)
        ngrp = ccnt // nb

        @pl.loop(0, ngrp)
        def _(grp):
            cb = (cstart + grp * nb) * ck
            ldr = [pltpu.async_copy(eidx_hbm.at[pl.ds(cb + b * ck, ck)],
                                    ridx.at[b], sem_ir.at[b])
                   for b in range(nb)]
            ldc = [pltpu.async_copy(eidx_hbm.at[pl.ds(e + cb + b * ck, ck)],
                                    cidx.at[b], sem_ic.at[b])
                   for b in range(nb)]
            gth = []
            for b in range(nb):
                ldr[b].wait()
                gth.append(pltpu.async_copy(g_hbm.at[ridx.at[b]], gbuf.at[b],
                                            sem_g.at[b]))
            sct = []
            for b in range(nb):
                gth[b].wait()
                ldc[b].wait()
                sct.append(pltpu.async_copy(gbuf.at[b], acc.at[cidx.at[b]],
                                            sem_s.at[b], add=True))
            for b in range(nb):
                sct[b].wait()

        @pl.loop(ngrp * nb, ccnt)
        def _(k):
            kb = (cstart + k) * ck
            pltpu.sync_copy(eidx_hbm.at[pl.ds(kb, ck)], ridx.at[0])
            pltpu.sync_copy(eidx_hbm.at[pl.ds(e + kb, ck)], cidx.at[0])
            pltpu.sync_copy(g_hbm.at[ridx.at[0]], gbuf.at[0])
            pltpu.sync_copy(gbuf.at[0], acc.at[cidx.at[0]], add=True)

        plsc.subcore_barrier()

        @pl.loop(0, nzc)
        def _(r):
            pltpu.sync_copy(acc.at[pl.ds(zb + r * zch, zch)],
                            s_hbm.at[pl.ds(cid * n + zb + r * zch, zch)])

    return agg_kernel(eidx, g)


_BN = 2000


def _tc_prolog(x, degp, n):
    din = x.shape[1]
    nb = n // _BN

    def body(x_ref, d0, d1, g_ref, dinv_ref):
        dinv = lax.rsqrt(d0[:, 0:1] + d1[:, 0:1] + 1.0)
        g_ref[...] = x_ref[...] * dinv
        dinv_ref[...] = dinv

    return pl.pallas_call(
        body,
        grid=(nb,),
        in_specs=[
            pl.BlockSpec((_BN, din), lambda i: (i, 0)),
            pl.BlockSpec((_BN, 128), lambda i: (i, 0)),
            pl.BlockSpec((_BN, 128), lambda i: (i + nb, 0)),
        ],
        out_specs=[
            pl.BlockSpec((_BN, din), lambda i: (i, 0)),
            pl.BlockSpec((_BN, 1), lambda i: (i, 0)),
        ],
        out_shape=[
            jax.ShapeDtypeStruct((n, din), _F32),
            jax.ShapeDtypeStruct((n, 1), _F32),
        ],
    )(x, degp, degp)


def _tc_layer1(s, g, dinv, w, b, n):
    din = g.shape[1]
    dout = w.shape[1]
    ho = dout // 2
    nb = n // _BN

    def body(p0, p1, g0, dv, w_ref, b_ref, out):
        t = (p0[...] + p1[...] + g0[...]) * dv[...]
        hh = jnp.dot(t, w_ref[...], precision=_HI,
                     preferred_element_type=_F32) + b_ref[...]
        hh = jnp.maximum(hh, 0.0)
        gg = hh * dv[...]
        out[0] = gg[:, :ho]
        out[1] = gg[:, ho:]

    return pl.pallas_call(
        body,
        grid=(nb,),
        in_specs=[
            pl.BlockSpec((_BN, din), lambda i: (i, 0)),
            pl.BlockSpec((_BN, din), lambda i: (i + nb, 0)),
            pl.BlockSpec((_BN, din), lambda i: (i, 0)),
            pl.BlockSpec((_BN, 1), lambda i: (i, 0)),
            pl.BlockSpec(w.shape, lambda i: (0, 0)),
            pl.BlockSpec(b.shape, lambda i: (0, 0)),
        ],
        out_specs=pl.BlockSpec((2, _BN, ho), lambda i: (0, i, 0)),
        out_shape=jax.ShapeDtypeStruct((2, n, ho), _F32),
    )(s, s, g, dinv, w, b)


def _tc_layer(s, g, dinv, w, b, n, w2=None):
    hf = s.shape[1]
    dout = (w2 if w2 is not None else w).shape[1]
    ho = dout // 2
    nb = n // _BN

    def body(s0, s1, g0, g1, dv, w_ref, b_ref, *rest):
        if w2 is not None:
            w2_ref, out = rest
        else:
            (out,) = rest
        t = jnp.concatenate([s0[...] + g0[...], s1[...] + g1[...]], axis=1)
        t = t * dv[...]
        hh = jnp.dot(t, w_ref[...], precision=_HI,
                     preferred_element_type=_F32) + b_ref[...]
        hh = jnp.maximum(hh, 0.0)
        if w2 is not None:
            hh = jnp.dot(hh, w2_ref[...], precision=_HI,
                         preferred_element_type=_F32)
        gg = hh * dv[...]
        out[0] = gg[:, :ho]
        out[1] = gg[:, ho:]

    in_specs = [
        pl.BlockSpec((_BN, hf), lambda i: (i, 0)),
        pl.BlockSpec((_BN, hf), lambda i: (i + nb, 0)),
        pl.BlockSpec((_BN, hf), lambda i: (i, 0)),
        pl.BlockSpec((_BN, hf), lambda i: (i + nb, 0)),
        pl.BlockSpec((_BN, 1), lambda i: (i, 0)),
        pl.BlockSpec(w.shape, lambda i: (0, 0)),
        pl.BlockSpec(b.shape, lambda i: (0, 0)),
    ]
    args = [s, s, g, g, dinv, w, b]
    if w2 is not None:
        in_specs.append(pl.BlockSpec(w2.shape, lambda i: (0, 0)))
        args.append(w2)
    return pl.pallas_call(
        body,
        grid=(nb,),
        in_specs=in_specs,
        out_specs=pl.BlockSpec((2, _BN, ho), lambda i: (0, i, 0)),
        out_shape=jax.ShapeDtypeStruct((2, n, ho), _F32),
    )(*args)


def _tc_epilog(s, g, dinv, b34, n):
    hf = s.shape[1]
    nb = n // _BN

    def body(s0, s1, g0, g1, dv, b_ref, out):
        t = jnp.concatenate([s0[...] + g0[...], s1[...] + g1[...]], axis=1)
        out[...] = t * dv[...] + b_ref[...]

    return pl.pallas_call(
        body,
        grid=(nb,),
        in_specs=[
            pl.BlockSpec((_BN, hf), lambda i: (i, 0)),
            pl.BlockSpec((_BN, hf), lambda i: (i + nb, 0)),
            pl.BlockSpec((_BN, hf), lambda i: (i, 0)),
            pl.BlockSpec((_BN, hf), lambda i: (i + nb, 0)),
            pl.BlockSpec((_BN, 1), lambda i: (i, 0)),
            pl.BlockSpec(b34.shape, lambda i: (0, 0)),
        ],
        out_specs=pl.BlockSpec((_BN, 2 * hf), lambda i: (i, 0)),
        out_shape=jax.ShapeDtypeStruct((n, 2 * hf), _F32),
    )(s, s, g, g, dinv, b34)


def _tc_fold(w3, wout, b3, bout):
    def body(w3_ref, wo_ref, b3_ref, bo_ref, w34_ref, b34_ref):
        w34_ref[...] = jnp.dot(w3_ref[...], wo_ref[...], precision=_HI,
                               preferred_element_type=_F32)
        b34_ref[...] = jnp.dot(b3_ref[...], wo_ref[...], precision=_HI,
                               preferred_element_type=_F32) + bo_ref[...]

    return pl.pallas_call(
        body,
        out_shape=[
            jax.ShapeDtypeStruct((w3.shape[0], wout.shape[1]), _F32),
            jax.ShapeDtypeStruct((1, wout.shape[1]), _F32),
        ],
    )(w3, wout, b3, bout)


def kernel(x, edge_index, W1, b1, W2, b2, W3, b3, Wout, bout):
    n, din = x.shape
    e = edge_index.shape[1]
    assert e % (2 * CHUNK) == 0
    eidx = edge_index.reshape(2 * e)

    b1r = b1.reshape(1, -1)
    b2r = b2.reshape(1, -1)
    b3r = b3.reshape(1, -1)
    boutr = bout.reshape(1, -1)

    w34, b34 = _tc_fold(W3, Wout, b3r, boutr)

    degp = _sc_deg(eidx, n)
    g1, dinv = _tc_prolog(x, degp, n)
    s1 = _sc_agg_es(eidx, g1, n)

    g2_3d = _tc_layer1(s1, g1, dinv, W1, b1r, n)
    g2 = g2_3d.reshape(2 * n, W1.shape[1] // 2)
    s2 = _sc_agg(eidx, g2, n, W1.shape[1] // 2, nb=3, ck=CHUNK)

    g3_3d = _tc_layer(s2, g2, dinv, W2, b2r, n, w2=w34)
    g3 = g3_3d.reshape(2 * n, Wout.shape[1] // 2)
    s3 = _sc_agg(eidx, g3, n, Wout.shape[1] // 2, ck=4 * CHUNK)

    return _tc_epilog(s3, g3, dinv, b34, n)
